# Initial kernel scaffold; baseline (speedup 1.0000x reference)
import jax
import jax.numpy as jnp
from jax.experimental import pallas as pl


def kernel(coords, edge_index, atom_types, density_table, density_deriv_table, pair_table, pair_deriv_table, embed_table, embed_deriv_table, embed_rho_min, embed_drho, cutoff, box_length):
    raise NotImplementedError("write your pallas kernel here")


# speedup vs baseline: 112.7987x; 112.7987x over previous
import functools

import jax
import jax.numpy as jnp
from jax import lax
from jax.experimental import pallas as pl
from jax.experimental.pallas import tpu as pltpu
from jax.experimental.pallas import tpu_sc as plsc

f32 = jnp.float32
i32 = jnp.int32

NC = 2
NS = 16
NW = NC * NS
LAN = 16
CHK = 512
GRP = CHK // 128

_SC_PARAMS = pltpu.CompilerParams(needs_layout_passes=False,
                                  use_tc_tiling_on_sc=False)


def _full(v, dtype=i32):
    return jnp.full((LAN,), v, dtype=dtype)


def _body_a1(per_w, atoms16, rowr, colr, params, geo,
             pv, row2d, col2d, arow, acol, sbuf, sem1, sem2):
    cid = lax.axis_index("c")
    sid = lax.axis_index("s")
    wid = cid * NS + sid
    pltpu.sync_copy(params, pv)
    iot = lax.iota(i32, LAN)
    lbox = pv[1]
    gpw = per_w // 128

    def chunk(t, carry):
        g0 = wid * gpw + t * GRP
        e0 = wid * per_w + t * CHK
        c1 = pltpu.async_copy(rowr.at[pl.ds(g0, GRP), :], row2d, sem1)
        c2 = pltpu.async_copy(colr.at[pl.ds(g0, GRP), :], col2d, sem1)
        c1.wait()
        c2.wait()
        descs = []
        for g in range(GRP):
            descs.append(pltpu.async_copy(
                atoms16.at[row2d.at[g]], arow.at[pl.ds(g * 128, 128), :], sem2))
            descs.append(pltpu.async_copy(
                atoms16.at[col2d.at[g]], acol.at[pl.ds(g * 128, 128), :], sem2))
        for d in descs:
            d.wait()

        def step(j, carry):
            lids = j * LAN + iot
            xr = plsc.load_gather(arow, [lids, _full(0)])
            yr = plsc.load_gather(arow, [lids, _full(1)])
            zr = plsc.load_gather(arow, [lids, _full(2)])
            tr = plsc.load_gather(arow, [lids, _full(3)])
            xc = plsc.load_gather(acol, [lids, _full(0)])
            yc = plsc.load_gather(acol, [lids, _full(1)])
            zc = plsc.load_gather(acol, [lids, _full(2)])
            tc = plsc.load_gather(acol, [lids, _full(3)])

            def mimage(d):
                u = d / lbox
                s = jnp.where(u > 0.5, 1.0, jnp.where(u < -0.5, -1.0, 0.0))
                return d - s * lbox

            plsc.store_scatter(sbuf, [_full(0), lids], mimage(xr - xc))
            plsc.store_scatter(sbuf, [_full(1), lids], mimage(yr - yc))
            plsc.store_scatter(sbuf, [_full(2), lids], mimage(zr - zc))
            plsc.store_scatter(sbuf, [_full(3), lids], tr)
            plsc.store_scatter(sbuf, [_full(4), lids], tc)
            return carry

        carry = lax.fori_loop(0, CHK // LAN, step, carry)
        pltpu.sync_copy(sbuf, geo.at[:, pl.ds(e0, CHK)])
        return carry

    lax.fori_loop(0, per_w // CHK, chunk, jnp.zeros((LAN,), f32))


def _tc_r_body(geo_ref, r_ref):
    dx = geo_ref[0, :]
    dy = geo_ref[1, :]
    dz = geo_ref[2, :]
    r_ref[0, :] = jnp.sqrt(dx * dx + dy * dy + dz * dz + 1e-12)


def _body_a2(n_r, nt, per_w, geo, r_all, rowr, dens, densd, pairf, pairdf,
             params, zeros1, staged, rho_part, epair_part,
             tab_d, tab_dd, tab_p, tab_pd, pv, row2d, gbuf, rbuf,
             sbuf, fcolb, ev, sh_rho, sem1, sem2):
    cid = lax.axis_index("c")
    sid = lax.axis_index("s")
    wid = cid * NS + sid
    npad = rho_part.shape[0] // NC
    sl1 = npad // NS

    pltpu.sync_copy(dens, tab_d)
    pltpu.sync_copy(densd, tab_dd)
    pltpu.sync_copy(pairf, tab_p)
    pltpu.sync_copy(pairdf, tab_pd)
    pltpu.sync_copy(params, pv)
    pltpu.sync_copy(zeros1, sh_rho.at[pl.ds(sid * sl1, sl1)])
    plsc.subcore_barrier()

    iot = lax.iota(i32, LAN)
    cutv = pv[0]
    invdr = pv[2]
    rclip = pv[3]
    n_chunks = per_w // CHK
    gpw = per_w // 128

    def chunk(t, acc):
        g0 = wid * gpw + t * GRP
        e0 = wid * per_w + t * CHK
        c1 = pltpu.async_copy(rowr.at[pl.ds(g0, GRP), :], row2d, sem1)
        c2 = pltpu.async_copy(geo.at[:, pl.ds(e0, CHK)], gbuf, sem1)
        c3 = pltpu.async_copy(r_all.at[0, pl.ds(e0, CHK)], rbuf, sem1)
        c1.wait()
        c2.wait()
        c3.wait()

        def step(j, acc):
            lids = j * LAN + iot
            dx = plsc.load_gather(gbuf, [_full(0), lids])
            dy = plsc.load_gather(gbuf, [_full(1), lids])
            dz = plsc.load_gather(gbuf, [_full(2), lids])
            tr = plsc.load_gather(gbuf, [_full(3), lids])
            tc = plsc.load_gather(gbuf, [_full(4), lids])
            r = plsc.load_gather(rbuf, [lids])
            wl = (r < cutv) & (r > 1e-6)
            wf = jnp.where(wl, 1.0, 0.0)
            rc = jnp.minimum(r, rclip)
            fidx = rc * invdr
            idxi = fidx.astype(i32)
            frac = fidx - idxi.astype(f32)
            nidxi = jnp.minimum(idxi + 1, n_r - 1)
            tci = tc.astype(i32)
            tri = tr.astype(i32)
            dbc = tci * n_r
            dbr = tri * n_r
            pb = (tri * nt + tci) * n_r

            v0 = plsc.load_gather(tab_d, [dbc + idxi])
            v1 = plsc.load_gather(tab_d, [dbc + nidxi])
            fcol = (v0 + frac * (v1 - v0)) * wf
            p0 = plsc.load_gather(tab_p, [pb + idxi])
            p1 = plsc.load_gather(tab_p, [pb + nidxi])
            acc = acc + (p0 + frac * (p1 - p0)) * wf
            q0 = plsc.load_gather(tab_pd, [pb + idxi])
            q1 = plsc.load_gather(tab_pd, [pb + nidxi])
            cval = (q0 + frac * (q1 - q0)) * wf
            a0 = plsc.load_gather(tab_dd, [dbc + idxi])
            a1 = plsc.load_gather(tab_dd, [dbc + nidxi])
            aval = (a0 + frac * (a1 - a0)) * wf
            b0 = plsc.load_gather(tab_dd, [dbr + idxi])
            b1 = plsc.load_gather(tab_dd, [dbr + nidxi])
            bval = (b0 + frac * (b1 - b0)) * wf
            rs = jnp.where(r > 1e-6, r, 1.0)
            plsc.store_scatter(sbuf, [_full(0), lids], aval)
            plsc.store_scatter(sbuf, [_full(1), lids], bval)
            plsc.store_scatter(sbuf, [_full(2), lids], cval)
            plsc.store_scatter(sbuf, [_full(3), lids], dx / rs)
            plsc.store_scatter(sbuf, [_full(4), lids], dy / rs)
            plsc.store_scatter(sbuf, [_full(5), lids], dz / rs)
            plsc.store_scatter(sbuf, [_full(6), lids], r)
            plsc.store_scatter(fcolb, [lids], fcol)
            return acc

        acc = lax.fori_loop(0, CHK // LAN, step, acc)
        wcp = pltpu.async_copy(sbuf, staged.at[:, pl.ds(e0, CHK)], sem1)
        for g in range(GRP):
            pltpu.sync_copy(fcolb.at[pl.ds(g * 128, 128)],
                            sh_rho.at[row2d.at[g]], add=True)
        wcp.wait()
        return acc

    acc = lax.fori_loop(0, n_chunks, chunk, jnp.zeros((LAN,), f32))
    plsc.subcore_barrier()
    pltpu.sync_copy(sh_rho.at[pl.ds(sid * sl1, sl1)],
                    rho_part.at[pl.ds(cid * npad + sid * sl1, sl1)])
    ev[...] = acc
    pltpu.sync_copy(ev, epair_part.at[pl.ds(wid * 16, 16)])


def _body_b(n_rho, n_real, rho_part, types_pad, emb, embd, rm16, dr16,
            fp_out, eemb_part, tab_e, tab_ed, r0b, r1b, tyb, fpb,
            rmv, drv, ev, sem):
    cid = lax.axis_index("c")
    sid = lax.axis_index("s")
    wid = cid * NS + sid
    npad = fp_out.shape[0]
    apw = npad // NW
    s0 = wid * apw

    pltpu.sync_copy(emb, tab_e)
    pltpu.sync_copy(embd, tab_ed)
    pltpu.sync_copy(rm16, rmv)
    pltpu.sync_copy(dr16, drv)
    pltpu.sync_copy(rho_part.at[pl.ds(s0, apw)], r0b)
    pltpu.sync_copy(rho_part.at[pl.ds(npad + s0, apw)], r1b)
    pltpu.sync_copy(types_pad.at[pl.ds(s0, apw)], tyb)

    iot = lax.iota(i32, LAN)
    khi = float((n_rho - 1) * (1.0 - 1e-07))

    def step(j, acc):
        lids = j * LAN + iot
        rho = plsc.load_gather(r0b, [lids]) + plsc.load_gather(r1b, [lids])
        t = plsc.load_gather(tyb, [lids])
        rm = plsc.load_gather(rmv, [t])
        dr = plsc.load_gather(drv, [t])
        hi = rm + dr * khi
        rhoc = jnp.minimum(jnp.maximum(rho, rm), hi)
        rf = (rhoc - rm) / dr
        ri = rf.astype(i32)
        rfr = rf - ri.astype(f32)
        rn = jnp.minimum(ri + 1, n_rho - 1)
        eb = t * n_rho
        F0 = plsc.load_gather(tab_e, [eb + ri])
        F1 = plsc.load_gather(tab_e, [eb + rn])
        G0 = plsc.load_gather(tab_ed, [eb + ri])
        G1 = plsc.load_gather(tab_ed, [eb + rn])
        fpv = G0 + rfr * (G1 - G0)
        plsc.store_scatter(fpb, [lids], fpv)
        m = jnp.where(s0 + lids < n_real, 1.0, 0.0)
        return acc + (F0 + rfr * (F1 - F0)) * m

    acc = lax.fori_loop(0, apw // LAN, step, jnp.zeros((LAN,), f32))
    pltpu.sync_copy(fpb, fp_out.at[pl.ds(s0, apw)])
    ev[...] = acc
    pltpu.sync_copy(ev, eemb_part.at[pl.ds(wid * 16, 16)])


def _body_c(per_w, staged, rowr, colr, fp_in, zeros1, f_part, vir_part,
            st, row2d, col2d, fprb, fpcb, fxb, fyb, fzb, ev,
            sh_fx, sh_fy, sh_fz, sem1, sem2):
    cid = lax.axis_index("c")
    sid = lax.axis_index("s")
    wid = cid * NS + sid
    npad = f_part.shape[0] // (NC * 3)
    sl1 = npad // NS

    pltpu.sync_copy(zeros1, sh_fx.at[pl.ds(sid * sl1, sl1)])
    pltpu.sync_copy(zeros1, sh_fy.at[pl.ds(sid * sl1, sl1)])
    pltpu.sync_copy(zeros1, sh_fz.at[pl.ds(sid * sl1, sl1)])
    plsc.subcore_barrier()

    iot = lax.iota(i32, LAN)
    n_chunks = per_w // CHK
    gpw = per_w // 128

    def chunk(t, acc):
        g0 = wid * gpw + t * GRP
        e0 = wid * per_w + t * CHK
        c1 = pltpu.async_copy(rowr.at[pl.ds(g0, GRP), :], row2d, sem1)
        c2 = pltpu.async_copy(colr.at[pl.ds(g0, GRP), :], col2d, sem1)
        c1.wait()
        c2.wait()
        descs = [pltpu.async_copy(staged.at[:, pl.ds(e0, CHK)], st, sem1)]
        for g in range(GRP):
            descs.append(pltpu.async_copy(
                fp_in.at[row2d.at[g]], fprb.at[pl.ds(g * 128, 128)], sem2))
            descs.append(pltpu.async_copy(
                fp_in.at[col2d.at[g]], fpcb.at[pl.ds(g * 128, 128)], sem2))
        for d in descs:
            d.wait()

        def step(j, acc):
            lids = j * LAN + iot
            fpr = plsc.load_gather(fprb, [lids])
            fpc = plsc.load_gather(fpcb, [lids])
            a = plsc.load_gather(st, [_full(0), lids])
            b = plsc.load_gather(st, [_full(1), lids])
            cc = plsc.load_gather(st, [_full(2), lids])
            gx = plsc.load_gather(st, [_full(3), lids])
            gy = plsc.load_gather(st, [_full(4), lids])
            gz = plsc.load_gather(st, [_full(5), lids])
            r = plsc.load_gather(st, [_full(6), lids])
            fpair = fpr * a + fpc * b + cc
            acc = acc + fpair * r
            plsc.store_scatter(fxb, [lids], -(fpair * gx))
            plsc.store_scatter(fyb, [lids], -(fpair * gy))
            plsc.store_scatter(fzb, [lids], -(fpair * gz))
            return acc

        acc = lax.fori_loop(0, CHK // LAN, step, acc)
        for g in range(GRP):
            pltpu.sync_copy(fxb.at[pl.ds(g * 128, 128)],
                            sh_fx.at[row2d.at[g]], add=True)
            pltpu.sync_copy(fyb.at[pl.ds(g * 128, 128)],
                            sh_fy.at[row2d.at[g]], add=True)
            pltpu.sync_copy(fzb.at[pl.ds(g * 128, 128)],
                            sh_fz.at[row2d.at[g]], add=True)
        return acc

    acc = lax.fori_loop(0, n_chunks, chunk, jnp.zeros((LAN,), f32))
    plsc.subcore_barrier()
    base = cid * 3 * npad + sid * sl1
    pltpu.sync_copy(sh_fx.at[pl.ds(sid * sl1, sl1)],
                    f_part.at[pl.ds(base, sl1)])
    pltpu.sync_copy(sh_fy.at[pl.ds(sid * sl1, sl1)],
                    f_part.at[pl.ds(base + npad, sl1)])
    pltpu.sync_copy(sh_fz.at[pl.ds(sid * sl1, sl1)],
                    f_part.at[pl.ds(base + 2 * npad, sl1)])
    ev[...] = acc
    pltpu.sync_copy(ev, vir_part.at[pl.ds(wid * 16, 16)])


def kernel(coords, edge_index, atom_types, density_table, density_deriv_table,
           pair_table, pair_deriv_table, embed_table, embed_deriv_table,
           embed_rho_min, embed_drho, cutoff, box_length):
    N = coords.shape[0]
    E = edge_index.shape[1]
    NT = density_table.shape[0]
    n_r = density_table.shape[1]
    n_rho = embed_table.shape[1]

    per_w = -(-E // (NW * CHK)) * CHK
    e_pad = NW * per_w
    npad = -(-N // (NW * 8)) * (NW * 8)
    sl1 = npad // NS

    cut = jnp.asarray(cutoff, f32)
    lbox = jnp.asarray(box_length, f32)
    inv_dr = 1.0 / (cut / (n_r - 1))
    rclip = cut * (1.0 - 1e-07)
    params = jnp.tile(jnp.stack([cut, lbox, inv_dr, rclip])[:, None],
                      (1, 16)).astype(f32)

    atoms16 = jnp.concatenate([coords, atom_types[:, None].astype(f32)], axis=1)
    atoms16 = jnp.pad(atoms16, ((0, npad - N), (0, 12)))
    rowr = jnp.pad(edge_index[0], (0, e_pad - E)).reshape(e_pad // 128, 128)
    colr = jnp.pad(edge_index[1], (0, e_pad - E)).reshape(e_pad // 128, 128)
    dens = density_table.reshape(-1)
    densd = density_deriv_table.reshape(-1)
    pairf = pair_table.reshape(-1)
    pairdf = pair_deriv_table.reshape(-1)
    emb = embed_table.reshape(-1)
    embd = embed_deriv_table.reshape(-1)
    rm16 = jnp.zeros((16,), f32).at[:NT].set(embed_rho_min)
    dr16 = jnp.ones((16,), f32).at[:NT].set(embed_drho)
    types_pad = jnp.pad(atom_types, (0, npad - N))
    zeros1 = jnp.zeros((sl1,), f32)

    mesh = plsc.VectorSubcoreMesh(core_axis_name="c", subcore_axis_name="s")

    ka1 = pl.kernel(
        functools.partial(_body_a1, per_w),
        out_type=jax.ShapeDtypeStruct((5, e_pad), f32),
        mesh=mesh,
        compiler_params=_SC_PARAMS,
        scratch_types=[
            pltpu.VMEM((4, 16), f32),
            pltpu.VMEM((GRP, 128), i32),
            pltpu.VMEM((GRP, 128), i32),
            pltpu.VMEM((CHK, 16), f32),
            pltpu.VMEM((CHK, 16), f32),
            pltpu.VMEM((5, CHK), f32),
            pltpu.SemaphoreType.DMA,
            pltpu.SemaphoreType.DMA,
        ],
    )
    geo = ka1(atoms16, rowr, colr, params)

    blk = 32768
    while e_pad % blk:
        blk //= 2
    r_all = pl.pallas_call(
        _tc_r_body,
        grid=(e_pad // blk,),
        in_specs=[pl.BlockSpec((5, blk), lambda i: (0, i))],
        out_specs=pl.BlockSpec((1, blk), lambda i: (0, i)),
        out_shape=jax.ShapeDtypeStruct((1, e_pad), f32),
    )(geo)

    ka2 = pl.kernel(
        functools.partial(_body_a2, n_r, NT, per_w),
        out_type=(
            jax.ShapeDtypeStruct((7, e_pad), f32),
            jax.ShapeDtypeStruct((NC * npad,), f32),
            jax.ShapeDtypeStruct((NW * 16,), f32),
        ),
        mesh=mesh,
        compiler_params=_SC_PARAMS,
        scratch_types=[
            pltpu.VMEM((NT * n_r,), f32),
            pltpu.VMEM((NT * n_r,), f32),
            pltpu.VMEM((NT * NT * n_r,), f32),
            pltpu.VMEM((NT * NT * n_r,), f32),
            pltpu.VMEM((4, 16), f32),
            pltpu.VMEM((GRP, 128), i32),
            pltpu.VMEM((5, CHK), f32),
            pltpu.VMEM((CHK,), f32),
            pltpu.VMEM((7, CHK), f32),
            pltpu.VMEM((CHK,), f32),
            pltpu.VMEM((16,), f32),
            pltpu.VMEM_SHARED((npad,), f32),
            pltpu.SemaphoreType.DMA,
            pltpu.SemaphoreType.DMA,
        ],
    )
    staged, rho_part, epair_part = ka2(geo, r_all, rowr, dens, densd,
                                       pairf, pairdf, params, zeros1)

    kb = pl.kernel(
        functools.partial(_body_b, n_rho, N),
        out_type=(
            jax.ShapeDtypeStruct((npad,), f32),
            jax.ShapeDtypeStruct((NW * 16,), f32),
        ),
        mesh=mesh,
        compiler_params=_SC_PARAMS,
        scratch_types=[
            pltpu.VMEM((NT * n_rho,), f32),
            pltpu.VMEM((NT * n_rho,), f32),
            pltpu.VMEM((npad // NW,), f32),
            pltpu.VMEM((npad // NW,), f32),
            pltpu.VMEM((npad // NW,), i32),
            pltpu.VMEM((npad // NW,), f32),
            pltpu.VMEM((16,), f32),
            pltpu.VMEM((16,), f32),
            pltpu.VMEM((16,), f32),
            pltpu.SemaphoreType.DMA,
        ],
    )
    fp_val, eemb_part = kb(rho_part, types_pad, emb, embd, rm16, dr16)

    kc = pl.kernel(
        functools.partial(_body_c, per_w),
        out_type=(
            jax.ShapeDtypeStruct((NC * 3 * npad,), f32),
            jax.ShapeDtypeStruct((NW * 16,), f32),
        ),
        mesh=mesh,
        compiler_params=_SC_PARAMS,
        scratch_types=[
            pltpu.VMEM((7, CHK), f32),
            pltpu.VMEM((GRP, 128), i32),
            pltpu.VMEM((GRP, 128), i32),
            pltpu.VMEM((CHK,), f32),
            pltpu.VMEM((CHK,), f32),
            pltpu.VMEM((CHK,), f32),
            pltpu.VMEM((CHK,), f32),
            pltpu.VMEM((CHK,), f32),
            pltpu.VMEM((16,), f32),
            pltpu.VMEM_SHARED((npad,), f32),
            pltpu.VMEM_SHARED((npad,), f32),
            pltpu.VMEM_SHARED((npad,), f32),
            pltpu.SemaphoreType.DMA,
            pltpu.SemaphoreType.DMA,
        ],
    )
    f_part, vir_part = kc(staged, rowr, colr, fp_val, zeros1)

    energy = jnp.sum(eemb_part) + 0.5 * jnp.sum(epair_part)
    virial = -0.5 * jnp.sum(vir_part)
    fp3 = f_part.reshape(NC, 3, npad)
    forces = (fp3[0] + fp3[1]).T[:N]
    return energy, forces, virial


# speedup vs baseline: 116.8928x; 1.0363x over previous
import functools

import jax
import jax.numpy as jnp
from jax import lax
from jax.experimental import pallas as pl
from jax.experimental.pallas import tpu as pltpu
from jax.experimental.pallas import tpu_sc as plsc

f32 = jnp.float32
i32 = jnp.int32

NC = 2
NS = 16
NW = NC * NS
LAN = 16
CHK = 512
GRP = CHK // 128

_SC_PARAMS = pltpu.CompilerParams(needs_layout_passes=False,
                                  use_tc_tiling_on_sc=False)


def _full(v, dtype=i32):
    return jnp.full((LAN,), v, dtype=dtype)


def _body_a1(per_w, atoms16, rowr, colr, params, geo,
             pv, row2d, col2d, arow, acol, sbuf, sem1, sem2):
    cid = lax.axis_index("c")
    sid = lax.axis_index("s")
    wid = cid * NS + sid
    pltpu.sync_copy(params, pv)
    iot = lax.iota(i32, LAN)
    lbox = pv[1]
    gpw = per_w // 128

    def chunk(t, carry):
        g0 = wid * gpw + t * GRP
        e0 = wid * per_w + t * CHK
        c1 = pltpu.async_copy(rowr.at[pl.ds(g0, GRP), :], row2d, sem1)
        c2 = pltpu.async_copy(colr.at[pl.ds(g0, GRP), :], col2d, sem1)
        c1.wait()
        c2.wait()
        descs = []
        for g in range(GRP):
            descs.append(pltpu.async_copy(
                atoms16.at[row2d.at[g]], arow.at[pl.ds(g * 128, 128), :], sem2))
            descs.append(pltpu.async_copy(
                atoms16.at[col2d.at[g]], acol.at[pl.ds(g * 128, 128), :], sem2))
        for d in descs:
            d.wait()

        def step(j4, carry):
          for q in range(4):
            j = j4 * 4 + q
            lids = j * LAN + iot
            xr = plsc.load_gather(arow, [lids, _full(0)])
            yr = plsc.load_gather(arow, [lids, _full(1)])
            zr = plsc.load_gather(arow, [lids, _full(2)])
            tr = plsc.load_gather(arow, [lids, _full(3)])
            xc = plsc.load_gather(acol, [lids, _full(0)])
            yc = plsc.load_gather(acol, [lids, _full(1)])
            zc = plsc.load_gather(acol, [lids, _full(2)])
            tc = plsc.load_gather(acol, [lids, _full(3)])

            def mimage(d):
                u = d / lbox
                s = jnp.where(u > 0.5, 1.0, jnp.where(u < -0.5, -1.0, 0.0))
                return d - s * lbox

            plsc.store_scatter(sbuf, [_full(0), lids], mimage(xr - xc))
            plsc.store_scatter(sbuf, [_full(1), lids], mimage(yr - yc))
            plsc.store_scatter(sbuf, [_full(2), lids], mimage(zr - zc))
            plsc.store_scatter(sbuf, [_full(3), lids], tr)
            plsc.store_scatter(sbuf, [_full(4), lids], tc)
          return carry

        carry = lax.fori_loop(0, CHK // LAN // 4, step, carry)
        pltpu.sync_copy(sbuf, geo.at[:, pl.ds(e0, CHK)])
        return carry

    lax.fori_loop(0, per_w // CHK, chunk, jnp.zeros((LAN,), f32))


def _tc_r_body(geo_ref, r_ref):
    dx = geo_ref[0, :]
    dy = geo_ref[1, :]
    dz = geo_ref[2, :]
    r_ref[0, :] = jnp.sqrt(dx * dx + dy * dy + dz * dz + 1e-12)


def _body_a2(n_r, nt, per_w, geo, r_all, rowr, dens, densd, pairf, pairdf,
             params, zeros1, staged, rho_part, epair_part,
             tab_d, tab_dd, tab_p, tab_pd, pv, row2d, gbuf, rbuf,
             sbuf, fcolb, ev, sh_rho, sem1, sem2):
    cid = lax.axis_index("c")
    sid = lax.axis_index("s")
    wid = cid * NS + sid
    npad = rho_part.shape[0] // NC
    sl1 = npad // NS

    pltpu.sync_copy(dens, tab_d)
    pltpu.sync_copy(densd, tab_dd)
    pltpu.sync_copy(pairf, tab_p)
    pltpu.sync_copy(pairdf, tab_pd)
    pltpu.sync_copy(params, pv)
    pltpu.sync_copy(zeros1, sh_rho.at[pl.ds(sid * sl1, sl1)])
    plsc.subcore_barrier()

    iot = lax.iota(i32, LAN)
    cutv = pv[0]
    invdr = pv[2]
    rclip = pv[3]
    n_chunks = per_w // CHK
    gpw = per_w // 128

    def chunk(t, acc):
        g0 = wid * gpw + t * GRP
        e0 = wid * per_w + t * CHK
        c1 = pltpu.async_copy(rowr.at[pl.ds(g0, GRP), :], row2d, sem1)
        c2 = pltpu.async_copy(geo.at[:, pl.ds(e0, CHK)], gbuf, sem1)
        c3 = pltpu.async_copy(r_all.at[0, pl.ds(e0, CHK)], rbuf, sem1)
        c1.wait()
        c2.wait()
        c3.wait()

        def step(j4, acc):
          for q in range(4):
            j = j4 * 4 + q
            lids = j * LAN + iot
            dx = plsc.load_gather(gbuf, [_full(0), lids])
            dy = plsc.load_gather(gbuf, [_full(1), lids])
            dz = plsc.load_gather(gbuf, [_full(2), lids])
            tr = plsc.load_gather(gbuf, [_full(3), lids])
            tc = plsc.load_gather(gbuf, [_full(4), lids])
            r = plsc.load_gather(rbuf, [lids])
            wl = (r < cutv) & (r > 1e-6)
            wf = jnp.where(wl, 1.0, 0.0)
            rc = jnp.minimum(r, rclip)
            fidx = rc * invdr
            idxi = fidx.astype(i32)
            frac = fidx - idxi.astype(f32)
            nidxi = jnp.minimum(idxi + 1, n_r - 1)
            tci = tc.astype(i32)
            tri = tr.astype(i32)
            dbc = tci * n_r
            dbr = tri * n_r
            pb = (tri * nt + tci) * n_r

            v0 = plsc.load_gather(tab_d, [dbc + idxi])
            v1 = plsc.load_gather(tab_d, [dbc + nidxi])
            fcol = (v0 + frac * (v1 - v0)) * wf
            p0 = plsc.load_gather(tab_p, [pb + idxi])
            p1 = plsc.load_gather(tab_p, [pb + nidxi])
            acc = acc + (p0 + frac * (p1 - p0)) * wf
            q0 = plsc.load_gather(tab_pd, [pb + idxi])
            q1 = plsc.load_gather(tab_pd, [pb + nidxi])
            cval = (q0 + frac * (q1 - q0)) * wf
            a0 = plsc.load_gather(tab_dd, [dbc + idxi])
            a1 = plsc.load_gather(tab_dd, [dbc + nidxi])
            aval = (a0 + frac * (a1 - a0)) * wf
            b0 = plsc.load_gather(tab_dd, [dbr + idxi])
            b1 = plsc.load_gather(tab_dd, [dbr + nidxi])
            bval = (b0 + frac * (b1 - b0)) * wf
            rs = jnp.where(r > 1e-6, r, 1.0)
            plsc.store_scatter(sbuf, [_full(0), lids], aval)
            plsc.store_scatter(sbuf, [_full(1), lids], bval)
            plsc.store_scatter(sbuf, [_full(2), lids], cval)
            plsc.store_scatter(sbuf, [_full(3), lids], dx / rs)
            plsc.store_scatter(sbuf, [_full(4), lids], dy / rs)
            plsc.store_scatter(sbuf, [_full(5), lids], dz / rs)
            plsc.store_scatter(sbuf, [_full(6), lids], r)
            plsc.store_scatter(fcolb, [lids], fcol)
          return acc

        acc = lax.fori_loop(0, CHK // LAN // 4, step, acc)
        wcp = pltpu.async_copy(sbuf, staged.at[:, pl.ds(e0, CHK)], sem1)
        adds = [pltpu.async_copy(fcolb.at[pl.ds(g * 128, 128)],
                                 sh_rho.at[row2d.at[g]], sem2, add=True)
                for g in range(GRP)]
        for d in adds:
            d.wait()
        wcp.wait()
        return acc

    acc = lax.fori_loop(0, n_chunks, chunk, jnp.zeros((LAN,), f32))
    plsc.subcore_barrier()
    pltpu.sync_copy(sh_rho.at[pl.ds(sid * sl1, sl1)],
                    rho_part.at[pl.ds(cid * npad + sid * sl1, sl1)])
    ev[...] = acc
    pltpu.sync_copy(ev, epair_part.at[pl.ds(wid * 16, 16)])


def _body_b(n_rho, n_real, rho_part, types_pad, emb, embd, rm16, dr16,
            fp_out, eemb_part, tab_e, tab_ed, r0b, r1b, tyb, fpb,
            rmv, drv, ev, sem):
    cid = lax.axis_index("c")
    sid = lax.axis_index("s")
    wid = cid * NS + sid
    npad = fp_out.shape[0]
    apw = npad // NW
    s0 = wid * apw

    pltpu.sync_copy(emb, tab_e)
    pltpu.sync_copy(embd, tab_ed)
    pltpu.sync_copy(rm16, rmv)
    pltpu.sync_copy(dr16, drv)
    pltpu.sync_copy(rho_part.at[pl.ds(s0, apw)], r0b)
    pltpu.sync_copy(rho_part.at[pl.ds(npad + s0, apw)], r1b)
    pltpu.sync_copy(types_pad.at[pl.ds(s0, apw)], tyb)

    iot = lax.iota(i32, LAN)
    khi = float((n_rho - 1) * (1.0 - 1e-07))

    def step(j, acc):
        lids = j * LAN + iot
        rho = plsc.load_gather(r0b, [lids]) + plsc.load_gather(r1b, [lids])
        t = plsc.load_gather(tyb, [lids])
        rm = plsc.load_gather(rmv, [t])
        dr = plsc.load_gather(drv, [t])
        hi = rm + dr * khi
        rhoc = jnp.minimum(jnp.maximum(rho, rm), hi)
        rf = (rhoc - rm) / dr
        ri = rf.astype(i32)
        rfr = rf - ri.astype(f32)
        rn = jnp.minimum(ri + 1, n_rho - 1)
        eb = t * n_rho
        F0 = plsc.load_gather(tab_e, [eb + ri])
        F1 = plsc.load_gather(tab_e, [eb + rn])
        G0 = plsc.load_gather(tab_ed, [eb + ri])
        G1 = plsc.load_gather(tab_ed, [eb + rn])
        fpv = G0 + rfr * (G1 - G0)
        plsc.store_scatter(fpb, [lids], fpv)
        m = jnp.where(s0 + lids < n_real, 1.0, 0.0)
        return acc + (F0 + rfr * (F1 - F0)) * m

    acc = lax.fori_loop(0, apw // LAN, step, jnp.zeros((LAN,), f32))
    pltpu.sync_copy(fpb, fp_out.at[pl.ds(s0, apw)])
    ev[...] = acc
    pltpu.sync_copy(ev, eemb_part.at[pl.ds(wid * 16, 16)])


def _body_c(per_w, staged, rowr, colr, fp_in, zeros1, f_part, vir_part,
            st, row2d, col2d, fprb, fpcb, fxb, fyb, fzb, ev,
            sh_fx, sh_fy, sh_fz, sem1, sem2):
    cid = lax.axis_index("c")
    sid = lax.axis_index("s")
    wid = cid * NS + sid
    npad = f_part.shape[0] // (NC * 3)
    sl1 = npad // NS

    pltpu.sync_copy(zeros1, sh_fx.at[pl.ds(sid * sl1, sl1)])
    pltpu.sync_copy(zeros1, sh_fy.at[pl.ds(sid * sl1, sl1)])
    pltpu.sync_copy(zeros1, sh_fz.at[pl.ds(sid * sl1, sl1)])
    plsc.subcore_barrier()

    iot = lax.iota(i32, LAN)
    n_chunks = per_w // CHK
    gpw = per_w // 128

    def chunk(t, acc):
        g0 = wid * gpw + t * GRP
        e0 = wid * per_w + t * CHK
        c1 = pltpu.async_copy(rowr.at[pl.ds(g0, GRP), :], row2d, sem1)
        c2 = pltpu.async_copy(colr.at[pl.ds(g0, GRP), :], col2d, sem1)
        c1.wait()
        c2.wait()
        descs = [pltpu.async_copy(staged.at[:, pl.ds(e0, CHK)], st, sem1)]
        for g in range(GRP):
            descs.append(pltpu.async_copy(
                fp_in.at[row2d.at[g]], fprb.at[pl.ds(g * 128, 128)], sem2))
            descs.append(pltpu.async_copy(
                fp_in.at[col2d.at[g]], fpcb.at[pl.ds(g * 128, 128)], sem2))
        for d in descs:
            d.wait()

        def step(j4, acc):
          for q in range(4):
            j = j4 * 4 + q
            lids = j * LAN + iot
            fpr = plsc.load_gather(fprb, [lids])
            fpc = plsc.load_gather(fpcb, [lids])
            a = plsc.load_gather(st, [_full(0), lids])
            b = plsc.load_gather(st, [_full(1), lids])
            cc = plsc.load_gather(st, [_full(2), lids])
            gx = plsc.load_gather(st, [_full(3), lids])
            gy = plsc.load_gather(st, [_full(4), lids])
            gz = plsc.load_gather(st, [_full(5), lids])
            r = plsc.load_gather(st, [_full(6), lids])
            fpair = fpr * a + fpc * b + cc
            acc = acc + fpair * r
            plsc.store_scatter(fxb, [lids], -(fpair * gx))
            plsc.store_scatter(fyb, [lids], -(fpair * gy))
            plsc.store_scatter(fzb, [lids], -(fpair * gz))
          return acc

        acc = lax.fori_loop(0, CHK // LAN // 4, step, acc)
        adds = []
        for g in range(GRP):
            adds.append(pltpu.async_copy(fxb.at[pl.ds(g * 128, 128)],
                                         sh_fx.at[row2d.at[g]], sem2, add=True))
            adds.append(pltpu.async_copy(fyb.at[pl.ds(g * 128, 128)],
                                         sh_fy.at[row2d.at[g]], sem2, add=True))
            adds.append(pltpu.async_copy(fzb.at[pl.ds(g * 128, 128)],
                                         sh_fz.at[row2d.at[g]], sem2, add=True))
        for d in adds:
            d.wait()
        return acc

    acc = lax.fori_loop(0, n_chunks, chunk, jnp.zeros((LAN,), f32))
    plsc.subcore_barrier()
    base = cid * 3 * npad + sid * sl1
    pltpu.sync_copy(sh_fx.at[pl.ds(sid * sl1, sl1)],
                    f_part.at[pl.ds(base, sl1)])
    pltpu.sync_copy(sh_fy.at[pl.ds(sid * sl1, sl1)],
                    f_part.at[pl.ds(base + npad, sl1)])
    pltpu.sync_copy(sh_fz.at[pl.ds(sid * sl1, sl1)],
                    f_part.at[pl.ds(base + 2 * npad, sl1)])
    ev[...] = acc
    pltpu.sync_copy(ev, vir_part.at[pl.ds(wid * 16, 16)])


def kernel(coords, edge_index, atom_types, density_table, density_deriv_table,
           pair_table, pair_deriv_table, embed_table, embed_deriv_table,
           embed_rho_min, embed_drho, cutoff, box_length):
    N = coords.shape[0]
    E = edge_index.shape[1]
    NT = density_table.shape[0]
    n_r = density_table.shape[1]
    n_rho = embed_table.shape[1]

    per_w = -(-E // (NW * CHK)) * CHK
    e_pad = NW * per_w
    npad = -(-N // (NW * 8)) * (NW * 8)
    sl1 = npad // NS

    cut = jnp.asarray(cutoff, f32)
    lbox = jnp.asarray(box_length, f32)
    inv_dr = 1.0 / (cut / (n_r - 1))
    rclip = cut * (1.0 - 1e-07)
    params = jnp.tile(jnp.stack([cut, lbox, inv_dr, rclip])[:, None],
                      (1, 16)).astype(f32)

    atoms16 = jnp.concatenate([coords, atom_types[:, None].astype(f32)], axis=1)
    atoms16 = jnp.pad(atoms16, ((0, npad - N), (0, 12)))
    rowr = jnp.pad(edge_index[0], (0, e_pad - E)).reshape(e_pad // 128, 128)
    colr = jnp.pad(edge_index[1], (0, e_pad - E)).reshape(e_pad // 128, 128)
    dens = density_table.reshape(-1)
    densd = density_deriv_table.reshape(-1)
    pairf = pair_table.reshape(-1)
    pairdf = pair_deriv_table.reshape(-1)
    emb = embed_table.reshape(-1)
    embd = embed_deriv_table.reshape(-1)
    rm16 = jnp.zeros((16,), f32).at[:NT].set(embed_rho_min)
    dr16 = jnp.ones((16,), f32).at[:NT].set(embed_drho)
    types_pad = jnp.pad(atom_types, (0, npad - N))
    zeros1 = jnp.zeros((sl1,), f32)

    mesh = plsc.VectorSubcoreMesh(core_axis_name="c", subcore_axis_name="s")

    ka1 = pl.kernel(
        functools.partial(_body_a1, per_w),
        out_type=jax.ShapeDtypeStruct((5, e_pad), f32),
        mesh=mesh,
        compiler_params=_SC_PARAMS,
        scratch_types=[
            pltpu.VMEM((4, 16), f32),
            pltpu.VMEM((GRP, 128), i32),
            pltpu.VMEM((GRP, 128), i32),
            pltpu.VMEM((CHK, 16), f32),
            pltpu.VMEM((CHK, 16), f32),
            pltpu.VMEM((5, CHK), f32),
            pltpu.SemaphoreType.DMA,
            pltpu.SemaphoreType.DMA,
        ],
    )
    geo = ka1(atoms16, rowr, colr, params)

    blk = 32768
    while e_pad % blk:
        blk //= 2
    r_all = pl.pallas_call(
        _tc_r_body,
        grid=(e_pad // blk,),
        in_specs=[pl.BlockSpec((5, blk), lambda i: (0, i))],
        out_specs=pl.BlockSpec((1, blk), lambda i: (0, i)),
        out_shape=jax.ShapeDtypeStruct((1, e_pad), f32),
    )(geo)

    ka2 = pl.kernel(
        functools.partial(_body_a2, n_r, NT, per_w),
        out_type=(
            jax.ShapeDtypeStruct((7, e_pad), f32),
            jax.ShapeDtypeStruct((NC * npad,), f32),
            jax.ShapeDtypeStruct((NW * 16,), f32),
        ),
        mesh=mesh,
        compiler_params=_SC_PARAMS,
        scratch_types=[
            pltpu.VMEM((NT * n_r,), f32),
            pltpu.VMEM((NT * n_r,), f32),
            pltpu.VMEM((NT * NT * n_r,), f32),
            pltpu.VMEM((NT * NT * n_r,), f32),
            pltpu.VMEM((4, 16), f32),
            pltpu.VMEM((GRP, 128), i32),
            pltpu.VMEM((5, CHK), f32),
            pltpu.VMEM((CHK,), f32),
            pltpu.VMEM((7, CHK), f32),
            pltpu.VMEM((CHK,), f32),
            pltpu.VMEM((16,), f32),
            pltpu.VMEM_SHARED((npad,), f32),
            pltpu.SemaphoreType.DMA,
            pltpu.SemaphoreType.DMA,
        ],
    )
    staged, rho_part, epair_part = ka2(geo, r_all, rowr, dens, densd,
                                       pairf, pairdf, params, zeros1)

    kb = pl.kernel(
        functools.partial(_body_b, n_rho, N),
        out_type=(
            jax.ShapeDtypeStruct((npad,), f32),
            jax.ShapeDtypeStruct((NW * 16,), f32),
        ),
        mesh=mesh,
        compiler_params=_SC_PARAMS,
        scratch_types=[
            pltpu.VMEM((NT * n_rho,), f32),
            pltpu.VMEM((NT * n_rho,), f32),
            pltpu.VMEM((npad // NW,), f32),
            pltpu.VMEM((npad // NW,), f32),
            pltpu.VMEM((npad // NW,), i32),
            pltpu.VMEM((npad // NW,), f32),
            pltpu.VMEM((16,), f32),
            pltpu.VMEM((16,), f32),
            pltpu.VMEM((16,), f32),
            pltpu.SemaphoreType.DMA,
        ],
    )
    fp_val, eemb_part = kb(rho_part, types_pad, emb, embd, rm16, dr16)

    kc = pl.kernel(
        functools.partial(_body_c, per_w),
        out_type=(
            jax.ShapeDtypeStruct((NC * 3 * npad,), f32),
            jax.ShapeDtypeStruct((NW * 16,), f32),
        ),
        mesh=mesh,
        compiler_params=_SC_PARAMS,
        scratch_types=[
            pltpu.VMEM((7, CHK), f32),
            pltpu.VMEM((GRP, 128), i32),
            pltpu.VMEM((GRP, 128), i32),
            pltpu.VMEM((CHK,), f32),
            pltpu.VMEM((CHK,), f32),
            pltpu.VMEM((CHK,), f32),
            pltpu.VMEM((CHK,), f32),
            pltpu.VMEM((CHK,), f32),
            pltpu.VMEM((16,), f32),
            pltpu.VMEM_SHARED((npad,), f32),
            pltpu.VMEM_SHARED((npad,), f32),
            pltpu.VMEM_SHARED((npad,), f32),
            pltpu.SemaphoreType.DMA,
            pltpu.SemaphoreType.DMA,
        ],
    )
    f_part, vir_part = kc(staged, rowr, colr, fp_val, zeros1)

    energy = jnp.sum(eemb_part) + 0.5 * jnp.sum(epair_part)
    virial = -0.5 * jnp.sum(vir_part)
    fp3 = f_part.reshape(NC, 3, npad)
    forces = (fp3[0] + fp3[1]).T[:N]
    return energy, forces, virial


# speedup vs baseline: 125.3849x; 1.0726x over previous
import functools

import jax
import jax.numpy as jnp
from jax import lax
from jax.experimental import pallas as pl
from jax.experimental.pallas import tpu as pltpu
from jax.experimental.pallas import tpu_sc as plsc

f32 = jnp.float32
i32 = jnp.int32

NC = 2
NS = 16
NW = NC * NS
LAN = 16
CHK = 1024
GRP = CHK // 128

_SC_PARAMS = pltpu.CompilerParams(needs_layout_passes=False,
                                  use_tc_tiling_on_sc=False)


def _full(v, dtype=i32):
    return jnp.full((LAN,), v, dtype=dtype)


def _body_a1(per_w, atoms16, rowr, colr, params, geo,
             pv, row2d, col2d, arow, acol, sbuf, sem1, sem2):
    cid = lax.axis_index("c")
    sid = lax.axis_index("s")
    wid = cid * NS + sid
    pltpu.sync_copy(params, pv)
    iot = lax.iota(i32, LAN)
    lbox = pv[1]
    gpw = per_w // 128

    def chunk(t, carry):
        g0 = wid * gpw + t * GRP
        e0 = wid * per_w + t * CHK
        c1 = pltpu.async_copy(rowr.at[pl.ds(g0, GRP), :], row2d, sem1)
        c2 = pltpu.async_copy(colr.at[pl.ds(g0, GRP), :], col2d, sem1)
        c1.wait()
        c2.wait()
        descs = []
        for g in range(GRP):
            descs.append(pltpu.async_copy(
                atoms16.at[row2d.at[g]], arow.at[pl.ds(g * 128, 128), :], sem2))
            descs.append(pltpu.async_copy(
                atoms16.at[col2d.at[g]], acol.at[pl.ds(g * 128, 128), :], sem2))
        for d in descs:
            d.wait()

        def step(j4, carry):
          for q in range(4):
            j = j4 * 4 + q
            lids = j * LAN + iot
            xr = plsc.load_gather(arow, [lids, _full(0)])
            yr = plsc.load_gather(arow, [lids, _full(1)])
            zr = plsc.load_gather(arow, [lids, _full(2)])
            tr = plsc.load_gather(arow, [lids, _full(3)])
            xc = plsc.load_gather(acol, [lids, _full(0)])
            yc = plsc.load_gather(acol, [lids, _full(1)])
            zc = plsc.load_gather(acol, [lids, _full(2)])
            tc = plsc.load_gather(acol, [lids, _full(3)])

            def mimage(d):
                u = d / lbox
                s = jnp.where(u > 0.5, 1.0, jnp.where(u < -0.5, -1.0, 0.0))
                return d - s * lbox

            plsc.store_scatter(sbuf, [_full(0), lids], mimage(xr - xc))
            plsc.store_scatter(sbuf, [_full(1), lids], mimage(yr - yc))
            plsc.store_scatter(sbuf, [_full(2), lids], mimage(zr - zc))
            plsc.store_scatter(sbuf, [_full(3), lids], tr)
            plsc.store_scatter(sbuf, [_full(4), lids], tc)
          return carry

        carry = lax.fori_loop(0, CHK // LAN // 4, step, carry)
        pltpu.sync_copy(sbuf, geo.at[:, pl.ds(e0, CHK)])
        return carry

    lax.fori_loop(0, per_w // CHK, chunk, jnp.zeros((LAN,), f32))


def _tc_r_body(geo_ref, r_ref):
    dx = geo_ref[0, :]
    dy = geo_ref[1, :]
    dz = geo_ref[2, :]
    r_ref[0, :] = jnp.sqrt(dx * dx + dy * dy + dz * dz + 1e-12)


def _body_a2(n_r, nt, per_w, geo, r_all, rowr, dens, densd, pairf, pairdf,
             params, zeros1, staged, rho_part, epair_part,
             tab_d, tab_dd, tab_p, tab_pd, pv, row2d, gbuf, rbuf,
             sbuf, fcolb, ev, sh_rho, sem1, sem2):
    cid = lax.axis_index("c")
    sid = lax.axis_index("s")
    wid = cid * NS + sid
    npad = rho_part.shape[0] // NC
    sl1 = npad // NS

    pltpu.sync_copy(dens, tab_d)
    pltpu.sync_copy(densd, tab_dd)
    pltpu.sync_copy(pairf, tab_p)
    pltpu.sync_copy(pairdf, tab_pd)
    pltpu.sync_copy(params, pv)
    pltpu.sync_copy(zeros1, sh_rho.at[pl.ds(sid * sl1, sl1)])
    plsc.subcore_barrier()

    iot = lax.iota(i32, LAN)
    cutv = pv[0]
    invdr = pv[2]
    rclip = pv[3]
    n_chunks = per_w // CHK
    gpw = per_w // 128

    def chunk(t, acc):
        g0 = wid * gpw + t * GRP
        e0 = wid * per_w + t * CHK
        c1 = pltpu.async_copy(rowr.at[pl.ds(g0, GRP), :], row2d, sem1)
        c2 = pltpu.async_copy(geo.at[:, pl.ds(e0, CHK)], gbuf, sem1)
        c3 = pltpu.async_copy(r_all.at[0, pl.ds(e0, CHK)], rbuf, sem1)
        c1.wait()
        c2.wait()
        c3.wait()

        def step(j4, acc):
          for q in range(4):
            j = j4 * 4 + q
            lids = j * LAN + iot
            dx = plsc.load_gather(gbuf, [_full(0), lids])
            dy = plsc.load_gather(gbuf, [_full(1), lids])
            dz = plsc.load_gather(gbuf, [_full(2), lids])
            tr = plsc.load_gather(gbuf, [_full(3), lids])
            tc = plsc.load_gather(gbuf, [_full(4), lids])
            r = plsc.load_gather(rbuf, [lids])
            wl = (r < cutv) & (r > 1e-6)
            wf = jnp.where(wl, 1.0, 0.0)
            rc = jnp.minimum(r, rclip)
            fidx = rc * invdr
            idxi = fidx.astype(i32)
            frac = fidx - idxi.astype(f32)
            nidxi = jnp.minimum(idxi + 1, n_r - 1)
            tci = tc.astype(i32)
            tri = tr.astype(i32)
            dbc = tci * n_r
            dbr = tri * n_r
            pb = (tri * nt + tci) * n_r

            v0 = plsc.load_gather(tab_d, [dbc + idxi])
            v1 = plsc.load_gather(tab_d, [dbc + nidxi])
            fcol = (v0 + frac * (v1 - v0)) * wf
            p0 = plsc.load_gather(tab_p, [pb + idxi])
            p1 = plsc.load_gather(tab_p, [pb + nidxi])
            acc = acc + (p0 + frac * (p1 - p0)) * wf
            q0 = plsc.load_gather(tab_pd, [pb + idxi])
            q1 = plsc.load_gather(tab_pd, [pb + nidxi])
            cval = (q0 + frac * (q1 - q0)) * wf
            a0 = plsc.load_gather(tab_dd, [dbc + idxi])
            a1 = plsc.load_gather(tab_dd, [dbc + nidxi])
            aval = (a0 + frac * (a1 - a0)) * wf
            b0 = plsc.load_gather(tab_dd, [dbr + idxi])
            b1 = plsc.load_gather(tab_dd, [dbr + nidxi])
            bval = (b0 + frac * (b1 - b0)) * wf
            rs = jnp.where(r > 1e-6, r, 1.0)
            plsc.store_scatter(sbuf, [_full(0), lids], aval)
            plsc.store_scatter(sbuf, [_full(1), lids], bval)
            plsc.store_scatter(sbuf, [_full(2), lids], cval)
            plsc.store_scatter(sbuf, [_full(3), lids], dx / rs)
            plsc.store_scatter(sbuf, [_full(4), lids], dy / rs)
            plsc.store_scatter(sbuf, [_full(5), lids], dz / rs)
            plsc.store_scatter(sbuf, [_full(6), lids], r)
            plsc.store_scatter(fcolb, [lids], fcol)
          return acc

        acc = lax.fori_loop(0, CHK // LAN // 4, step, acc)
        wcp = pltpu.async_copy(sbuf, staged.at[:, pl.ds(e0, CHK)], sem1)
        adds = [pltpu.async_copy(fcolb.at[pl.ds(g * 128, 128)],
                                 sh_rho.at[row2d.at[g]], sem2, add=True)
                for g in range(GRP)]
        for d in adds:
            d.wait()
        wcp.wait()
        return acc

    acc = lax.fori_loop(0, n_chunks, chunk, jnp.zeros((LAN,), f32))
    plsc.subcore_barrier()
    pltpu.sync_copy(sh_rho.at[pl.ds(sid * sl1, sl1)],
                    rho_part.at[pl.ds(cid * npad + sid * sl1, sl1)])
    ev[...] = acc
    pltpu.sync_copy(ev, epair_part.at[pl.ds(wid * 16, 16)])


def _body_b(n_rho, n_real, rho_part, types_pad, emb, embd, rm16, dr16,
            fp_out, eemb_part, tab_e, tab_ed, r0b, r1b, tyb, fpb,
            rmv, drv, ev, sem):
    cid = lax.axis_index("c")
    sid = lax.axis_index("s")
    wid = cid * NS + sid
    npad = fp_out.shape[0]
    apw = npad // NW
    s0 = wid * apw

    pltpu.sync_copy(emb, tab_e)
    pltpu.sync_copy(embd, tab_ed)
    pltpu.sync_copy(rm16, rmv)
    pltpu.sync_copy(dr16, drv)
    pltpu.sync_copy(rho_part.at[pl.ds(s0, apw)], r0b)
    pltpu.sync_copy(rho_part.at[pl.ds(npad + s0, apw)], r1b)
    pltpu.sync_copy(types_pad.at[pl.ds(s0, apw)], tyb)

    iot = lax.iota(i32, LAN)
    khi = float((n_rho - 1) * (1.0 - 1e-07))

    def step(j, acc):
        lids = j * LAN + iot
        rho = plsc.load_gather(r0b, [lids]) + plsc.load_gather(r1b, [lids])
        t = plsc.load_gather(tyb, [lids])
        rm = plsc.load_gather(rmv, [t])
        dr = plsc.load_gather(drv, [t])
        hi = rm + dr * khi
        rhoc = jnp.minimum(jnp.maximum(rho, rm), hi)
        rf = (rhoc - rm) / dr
        ri = rf.astype(i32)
        rfr = rf - ri.astype(f32)
        rn = jnp.minimum(ri + 1, n_rho - 1)
        eb = t * n_rho
        F0 = plsc.load_gather(tab_e, [eb + ri])
        F1 = plsc.load_gather(tab_e, [eb + rn])
        G0 = plsc.load_gather(tab_ed, [eb + ri])
        G1 = plsc.load_gather(tab_ed, [eb + rn])
        fpv = G0 + rfr * (G1 - G0)
        plsc.store_scatter(fpb, [lids], fpv)
        m = jnp.where(s0 + lids < n_real, 1.0, 0.0)
        return acc + (F0 + rfr * (F1 - F0)) * m

    acc = lax.fori_loop(0, apw // LAN, step, jnp.zeros((LAN,), f32))
    pltpu.sync_copy(fpb, fp_out.at[pl.ds(s0, apw)])
    ev[...] = acc
    pltpu.sync_copy(ev, eemb_part.at[pl.ds(wid * 16, 16)])


def _body_c(per_w, staged, rowr, colr, fp_in, zeros1, f_part, vir_part,
            st, row2d, col2d, fprb, fpcb, fxb, fyb, fzb, ev,
            sh_fx, sh_fy, sh_fz, sem1, sem2):
    cid = lax.axis_index("c")
    sid = lax.axis_index("s")
    wid = cid * NS + sid
    npad = f_part.shape[0] // (NC * 3)
    sl1 = npad // NS

    pltpu.sync_copy(zeros1, sh_fx.at[pl.ds(sid * sl1, sl1)])
    pltpu.sync_copy(zeros1, sh_fy.at[pl.ds(sid * sl1, sl1)])
    pltpu.sync_copy(zeros1, sh_fz.at[pl.ds(sid * sl1, sl1)])
    plsc.subcore_barrier()

    iot = lax.iota(i32, LAN)
    n_chunks = per_w // CHK
    gpw = per_w // 128

    def chunk(t, acc):
        g0 = wid * gpw + t * GRP
        e0 = wid * per_w + t * CHK
        c1 = pltpu.async_copy(rowr.at[pl.ds(g0, GRP), :], row2d, sem1)
        c2 = pltpu.async_copy(colr.at[pl.ds(g0, GRP), :], col2d, sem1)
        c1.wait()
        c2.wait()
        descs = [pltpu.async_copy(staged.at[:, pl.ds(e0, CHK)], st, sem1)]
        for g in range(GRP):
            descs.append(pltpu.async_copy(
                fp_in.at[row2d.at[g]], fprb.at[pl.ds(g * 128, 128)], sem2))
            descs.append(pltpu.async_copy(
                fp_in.at[col2d.at[g]], fpcb.at[pl.ds(g * 128, 128)], sem2))
        for d in descs:
            d.wait()

        def step(j4, acc):
          for q in range(4):
            j = j4 * 4 + q
            lids = j * LAN + iot
            fpr = plsc.load_gather(fprb, [lids])
            fpc = plsc.load_gather(fpcb, [lids])
            a = plsc.load_gather(st, [_full(0), lids])
            b = plsc.load_gather(st, [_full(1), lids])
            cc = plsc.load_gather(st, [_full(2), lids])
            gx = plsc.load_gather(st, [_full(3), lids])
            gy = plsc.load_gather(st, [_full(4), lids])
            gz = plsc.load_gather(st, [_full(5), lids])
            r = plsc.load_gather(st, [_full(6), lids])
            fpair = fpr * a + fpc * b + cc
            acc = acc + fpair * r
            plsc.store_scatter(fxb, [lids], -(fpair * gx))
            plsc.store_scatter(fyb, [lids], -(fpair * gy))
            plsc.store_scatter(fzb, [lids], -(fpair * gz))
          return acc

        acc = lax.fori_loop(0, CHK // LAN // 4, step, acc)
        adds = []
        for g in range(GRP):
            adds.append(pltpu.async_copy(fxb.at[pl.ds(g * 128, 128)],
                                         sh_fx.at[row2d.at[g]], sem2, add=True))
            adds.append(pltpu.async_copy(fyb.at[pl.ds(g * 128, 128)],
                                         sh_fy.at[row2d.at[g]], sem2, add=True))
            adds.append(pltpu.async_copy(fzb.at[pl.ds(g * 128, 128)],
                                         sh_fz.at[row2d.at[g]], sem2, add=True))
        for d in adds:
            d.wait()
        return acc

    acc = lax.fori_loop(0, n_chunks, chunk, jnp.zeros((LAN,), f32))
    plsc.subcore_barrier()
    base = cid * 3 * npad + sid * sl1
    pltpu.sync_copy(sh_fx.at[pl.ds(sid * sl1, sl1)],
                    f_part.at[pl.ds(base, sl1)])
    pltpu.sync_copy(sh_fy.at[pl.ds(sid * sl1, sl1)],
                    f_part.at[pl.ds(base + npad, sl1)])
    pltpu.sync_copy(sh_fz.at[pl.ds(sid * sl1, sl1)],
                    f_part.at[pl.ds(base + 2 * npad, sl1)])
    ev[...] = acc
    pltpu.sync_copy(ev, vir_part.at[pl.ds(wid * 16, 16)])


def kernel(coords, edge_index, atom_types, density_table, density_deriv_table,
           pair_table, pair_deriv_table, embed_table, embed_deriv_table,
           embed_rho_min, embed_drho, cutoff, box_length):
    N = coords.shape[0]
    E = edge_index.shape[1]
    NT = density_table.shape[0]
    n_r = density_table.shape[1]
    n_rho = embed_table.shape[1]

    per_w = -(-E // (NW * CHK)) * CHK
    e_pad = NW * per_w
    npad = -(-N // (NW * 8)) * (NW * 8)
    sl1 = npad // NS

    cut = jnp.asarray(cutoff, f32)
    lbox = jnp.asarray(box_length, f32)
    inv_dr = 1.0 / (cut / (n_r - 1))
    rclip = cut * (1.0 - 1e-07)
    params = jnp.tile(jnp.stack([cut, lbox, inv_dr, rclip])[:, None],
                      (1, 16)).astype(f32)

    atoms16 = jnp.concatenate([coords, atom_types[:, None].astype(f32)], axis=1)
    atoms16 = jnp.pad(atoms16, ((0, npad - N), (0, 12)))
    rowr = jnp.pad(edge_index[0], (0, e_pad - E)).reshape(e_pad // 128, 128)
    colr = jnp.pad(edge_index[1], (0, e_pad - E)).reshape(e_pad // 128, 128)
    dens = density_table.reshape(-1)
    densd = density_deriv_table.reshape(-1)
    pairf = pair_table.reshape(-1)
    pairdf = pair_deriv_table.reshape(-1)
    emb = embed_table.reshape(-1)
    embd = embed_deriv_table.reshape(-1)
    rm16 = jnp.zeros((16,), f32).at[:NT].set(embed_rho_min)
    dr16 = jnp.ones((16,), f32).at[:NT].set(embed_drho)
    types_pad = jnp.pad(atom_types, (0, npad - N))
    zeros1 = jnp.zeros((sl1,), f32)

    mesh = plsc.VectorSubcoreMesh(core_axis_name="c", subcore_axis_name="s")

    ka1 = pl.kernel(
        functools.partial(_body_a1, per_w),
        out_type=jax.ShapeDtypeStruct((5, e_pad), f32),
        mesh=mesh,
        compiler_params=_SC_PARAMS,
        scratch_types=[
            pltpu.VMEM((4, 16), f32),
            pltpu.VMEM((GRP, 128), i32),
            pltpu.VMEM((GRP, 128), i32),
            pltpu.VMEM((CHK, 16), f32),
            pltpu.VMEM((CHK, 16), f32),
            pltpu.VMEM((5, CHK), f32),
            pltpu.SemaphoreType.DMA,
            pltpu.SemaphoreType.DMA,
        ],
    )
    geo = ka1(atoms16, rowr, colr, params)

    blk = 32768
    while e_pad % blk:
        blk //= 2
    r_all = pl.pallas_call(
        _tc_r_body,
        grid=(e_pad // blk,),
        in_specs=[pl.BlockSpec((5, blk), lambda i: (0, i))],
        out_specs=pl.BlockSpec((1, blk), lambda i: (0, i)),
        out_shape=jax.ShapeDtypeStruct((1, e_pad), f32),
    )(geo)

    ka2 = pl.kernel(
        functools.partial(_body_a2, n_r, NT, per_w),
        out_type=(
            jax.ShapeDtypeStruct((7, e_pad), f32),
            jax.ShapeDtypeStruct((NC * npad,), f32),
            jax.ShapeDtypeStruct((NW * 16,), f32),
        ),
        mesh=mesh,
        compiler_params=_SC_PARAMS,
        scratch_types=[
            pltpu.VMEM((NT * n_r,), f32),
            pltpu.VMEM((NT * n_r,), f32),
            pltpu.VMEM((NT * NT * n_r,), f32),
            pltpu.VMEM((NT * NT * n_r,), f32),
            pltpu.VMEM((4, 16), f32),
            pltpu.VMEM((GRP, 128), i32),
            pltpu.VMEM((5, CHK), f32),
            pltpu.VMEM((CHK,), f32),
            pltpu.VMEM((7, CHK), f32),
            pltpu.VMEM((CHK,), f32),
            pltpu.VMEM((16,), f32),
            pltpu.VMEM_SHARED((npad,), f32),
            pltpu.SemaphoreType.DMA,
            pltpu.SemaphoreType.DMA,
        ],
    )
    staged, rho_part, epair_part = ka2(geo, r_all, rowr, dens, densd,
                                       pairf, pairdf, params, zeros1)

    kb = pl.kernel(
        functools.partial(_body_b, n_rho, N),
        out_type=(
            jax.ShapeDtypeStruct((npad,), f32),
            jax.ShapeDtypeStruct((NW * 16,), f32),
        ),
        mesh=mesh,
        compiler_params=_SC_PARAMS,
        scratch_types=[
            pltpu.VMEM((NT * n_rho,), f32),
            pltpu.VMEM((NT * n_rho,), f32),
            pltpu.VMEM((npad // NW,), f32),
            pltpu.VMEM((npad // NW,), f32),
            pltpu.VMEM((npad // NW,), i32),
            pltpu.VMEM((npad // NW,), f32),
            pltpu.VMEM((16,), f32),
            pltpu.VMEM((16,), f32),
            pltpu.VMEM((16,), f32),
            pltpu.SemaphoreType.DMA,
        ],
    )
    fp_val, eemb_part = kb(rho_part, types_pad, emb, embd, rm16, dr16)

    kc = pl.kernel(
        functools.partial(_body_c, per_w),
        out_type=(
            jax.ShapeDtypeStruct((NC * 3 * npad,), f32),
            jax.ShapeDtypeStruct((NW * 16,), f32),
        ),
        mesh=mesh,
        compiler_params=_SC_PARAMS,
        scratch_types=[
            pltpu.VMEM((7, CHK), f32),
            pltpu.VMEM((GRP, 128), i32),
            pltpu.VMEM((GRP, 128), i32),
            pltpu.VMEM((CHK,), f32),
            pltpu.VMEM((CHK,), f32),
            pltpu.VMEM((CHK,), f32),
            pltpu.VMEM((CHK,), f32),
            pltpu.VMEM((CHK,), f32),
            pltpu.VMEM((16,), f32),
            pltpu.VMEM_SHARED((npad,), f32),
            pltpu.VMEM_SHARED((npad,), f32),
            pltpu.VMEM_SHARED((npad,), f32),
            pltpu.SemaphoreType.DMA,
            pltpu.SemaphoreType.DMA,
        ],
    )
    f_part, vir_part = kc(staged, rowr, colr, fp_val, zeros1)

    energy = jnp.sum(eemb_part) + 0.5 * jnp.sum(epair_part)
    virial = -0.5 * jnp.sum(vir_part)
    fp3 = f_part.reshape(NC, 3, npad)
    forces = (fp3[0] + fp3[1]).T[:N]
    return energy, forces, virial


# speedup vs baseline: 133.4489x; 1.0643x over previous
import functools

import jax
import jax.numpy as jnp
from jax import lax
from jax.experimental import pallas as pl
from jax.experimental.pallas import tpu as pltpu
from jax.experimental.pallas import tpu_sc as plsc

f32 = jnp.float32
i32 = jnp.int32

NC = 2
NS = 16
NW = NC * NS
LAN = 16
CHK = 1024
GRP = CHK // 128

_SC_PARAMS = pltpu.CompilerParams(needs_layout_passes=False,
                                  use_tc_tiling_on_sc=False)


def _full(v, dtype=i32):
    return jnp.full((LAN,), v, dtype=dtype)


def _body_a1(per_w, atoms16, rowr, colr, params, geo,
             pv, row2d, col2d, arow, acol, sbuf, sem1, sem2):
    cid = lax.axis_index("c")
    sid = lax.axis_index("s")
    wid = cid * NS + sid
    pltpu.sync_copy(params, pv)
    iot = lax.iota(i32, LAN)
    lbox = pv[1]
    gpw = per_w // 128

    def chunk(t, carry):
        g0 = wid * gpw + t * GRP
        e0 = wid * per_w + t * CHK
        c1 = pltpu.async_copy(rowr.at[pl.ds(g0, GRP), :], row2d, sem1)
        c2 = pltpu.async_copy(colr.at[pl.ds(g0, GRP), :], col2d, sem1)
        c1.wait()
        c2.wait()
        descs = []
        for g in range(GRP):
            descs.append(pltpu.async_copy(
                atoms16.at[row2d.at[g]], arow.at[pl.ds(g * 128, 128), :], sem2))
            descs.append(pltpu.async_copy(
                atoms16.at[col2d.at[g]], acol.at[pl.ds(g * 128, 128), :], sem2))
        for d in descs:
            d.wait()

        def step(j4, carry):
          for q in range(4):
            j = j4 * 4 + q
            lids = j * LAN + iot
            xr = plsc.load_gather(arow, [lids, _full(0)])
            yr = plsc.load_gather(arow, [lids, _full(1)])
            zr = plsc.load_gather(arow, [lids, _full(2)])
            tr = plsc.load_gather(arow, [lids, _full(3)])
            xc = plsc.load_gather(acol, [lids, _full(0)])
            yc = plsc.load_gather(acol, [lids, _full(1)])
            zc = plsc.load_gather(acol, [lids, _full(2)])
            tc = plsc.load_gather(acol, [lids, _full(3)])

            def mimage(d):
                u = d / lbox
                s = jnp.where(u > 0.5, 1.0, jnp.where(u < -0.5, -1.0, 0.0))
                return d - s * lbox

            plsc.store_scatter(sbuf, [_full(0), lids], mimage(xr - xc))
            plsc.store_scatter(sbuf, [_full(1), lids], mimage(yr - yc))
            plsc.store_scatter(sbuf, [_full(2), lids], mimage(zr - zc))
            plsc.store_scatter(sbuf, [_full(3), lids], tr)
            plsc.store_scatter(sbuf, [_full(4), lids], tc)
          return carry

        carry = lax.fori_loop(0, CHK // LAN // 4, step, carry)
        pltpu.sync_copy(sbuf, geo.at[:, pl.ds(e0, CHK)])
        return carry

    lax.fori_loop(0, per_w // CHK, chunk, jnp.zeros((LAN,), f32))


def _tc_r_body(geo_ref, r_ref):
    dx = geo_ref[0, :]
    dy = geo_ref[1, :]
    dz = geo_ref[2, :]
    r_ref[0, :] = jnp.sqrt(dx * dx + dy * dy + dz * dz + 1e-12)


def _body_a2(n_r, nt, per_w, geo, r_all, rowr, dens, densd, pairf, pairdf,
             params, zeros1, staged, rho_part, epair_part,
             tab_d, tab_dd, tab_p, tab_pd, pv, row2d, gbuf, rbuf,
             sbuf, fcolb, ev, sh_rho, sem1, sem2):
    cid = lax.axis_index("c")
    sid = lax.axis_index("s")
    wid = cid * NS + sid
    npad = rho_part.shape[0] // NC
    sl1 = npad // NS

    pltpu.sync_copy(dens, tab_d)
    pltpu.sync_copy(densd, tab_dd)
    pltpu.sync_copy(pairf, tab_p)
    pltpu.sync_copy(pairdf, tab_pd)
    pltpu.sync_copy(params, pv)
    pltpu.sync_copy(zeros1, sh_rho.at[pl.ds(sid * sl1, sl1)])
    plsc.subcore_barrier()

    iot = lax.iota(i32, LAN)
    cutv = pv[0]
    invdr = pv[2]
    rclip = pv[3]
    n_chunks = per_w // CHK
    gpw = per_w // 128

    def chunk(t, acc):
        g0 = wid * gpw + t * GRP
        e0 = wid * per_w + t * CHK
        c1 = pltpu.async_copy(rowr.at[pl.ds(g0, GRP), :], row2d, sem1)
        c2 = pltpu.async_copy(geo.at[:, pl.ds(e0, CHK)], gbuf, sem1)
        c3 = pltpu.async_copy(r_all.at[0, pl.ds(e0, CHK)], rbuf, sem1)
        c1.wait()
        c2.wait()
        c3.wait()

        def step(j4, acc):
          for q in range(4):
            j = j4 * 4 + q
            lids = j * LAN + iot
            dx = plsc.load_gather(gbuf, [_full(0), lids])
            dy = plsc.load_gather(gbuf, [_full(1), lids])
            dz = plsc.load_gather(gbuf, [_full(2), lids])
            tr = plsc.load_gather(gbuf, [_full(3), lids])
            tc = plsc.load_gather(gbuf, [_full(4), lids])
            r = plsc.load_gather(rbuf, [lids])
            wl = (r < cutv) & (r > 1e-6)
            wf = jnp.where(wl, 1.0, 0.0)
            rc = jnp.minimum(r, rclip)
            fidx = rc * invdr
            idxi = fidx.astype(i32)
            frac = fidx - idxi.astype(f32)
            nidxi = jnp.minimum(idxi + 1, n_r - 1)
            tci = tc.astype(i32)
            tri = tr.astype(i32)
            dbc = tci * n_r
            dbr = tri * n_r
            pb = (tri * nt + tci) * n_r

            v0 = plsc.load_gather(tab_d, [dbc + idxi])
            v1 = plsc.load_gather(tab_d, [dbc + nidxi])
            fcol = (v0 + frac * (v1 - v0)) * wf
            p0 = plsc.load_gather(tab_p, [pb + idxi])
            p1 = plsc.load_gather(tab_p, [pb + nidxi])
            acc = acc + (p0 + frac * (p1 - p0)) * wf
            q0 = plsc.load_gather(tab_pd, [pb + idxi])
            q1 = plsc.load_gather(tab_pd, [pb + nidxi])
            cval = (q0 + frac * (q1 - q0)) * wf
            a0 = plsc.load_gather(tab_dd, [dbc + idxi])
            a1 = plsc.load_gather(tab_dd, [dbc + nidxi])
            aval = (a0 + frac * (a1 - a0)) * wf
            b0 = plsc.load_gather(tab_dd, [dbr + idxi])
            b1 = plsc.load_gather(tab_dd, [dbr + nidxi])
            bval = (b0 + frac * (b1 - b0)) * wf
            rs = jnp.where(r > 1e-6, r, 1.0)
            plsc.store_scatter(sbuf, [_full(0), lids], aval)
            plsc.store_scatter(sbuf, [_full(1), lids], bval)
            plsc.store_scatter(sbuf, [_full(2), lids], cval)
            plsc.store_scatter(sbuf, [_full(3), lids], dx / rs)
            plsc.store_scatter(sbuf, [_full(4), lids], dy / rs)
            plsc.store_scatter(sbuf, [_full(5), lids], dz / rs)
            plsc.store_scatter(sbuf, [_full(6), lids], r)
            plsc.store_scatter(fcolb, [lids], fcol)
          return acc

        acc = lax.fori_loop(0, CHK // LAN // 4, step, acc)
        wcp = pltpu.async_copy(sbuf, staged.at[:, pl.ds(e0, CHK)], sem1)
        adds = [pltpu.async_copy(fcolb.at[pl.ds(g * 128, 128)],
                                 sh_rho.at[row2d.at[g]], sem2, add=True)
                for g in range(GRP)]
        for d in adds:
            d.wait()
        wcp.wait()
        return acc

    acc = lax.fori_loop(0, n_chunks, chunk, jnp.zeros((LAN,), f32))
    plsc.subcore_barrier()
    pltpu.sync_copy(sh_rho.at[pl.ds(sid * sl1, sl1)],
                    rho_part.at[pl.ds(cid * npad + sid * sl1, sl1)])
    ev[...] = acc
    pltpu.sync_copy(ev, epair_part.at[pl.ds(wid * 16, 16)])


def _body_b(n_rho, n_real, rho_part, types_pad, emb, embd, rm16, dr16,
            fp_out, eemb_part, tab_e, tab_ed, r0b, r1b, tyb, fpb,
            rmv, drv, ev, sem):
    cid = lax.axis_index("c")
    sid = lax.axis_index("s")
    wid = cid * NS + sid
    npad = fp_out.shape[0]
    apw = npad // NW
    s0 = wid * apw

    pltpu.sync_copy(emb, tab_e)
    pltpu.sync_copy(embd, tab_ed)
    pltpu.sync_copy(rm16, rmv)
    pltpu.sync_copy(dr16, drv)
    pltpu.sync_copy(rho_part.at[pl.ds(s0, apw)], r0b)
    pltpu.sync_copy(rho_part.at[pl.ds(npad + s0, apw)], r1b)
    pltpu.sync_copy(types_pad.at[pl.ds(s0, apw)], tyb)

    iot = lax.iota(i32, LAN)
    khi = float((n_rho - 1) * (1.0 - 1e-07))

    def step(j, acc):
        lids = j * LAN + iot
        rho = plsc.load_gather(r0b, [lids]) + plsc.load_gather(r1b, [lids])
        t = plsc.load_gather(tyb, [lids])
        rm = plsc.load_gather(rmv, [t])
        dr = plsc.load_gather(drv, [t])
        hi = rm + dr * khi
        rhoc = jnp.minimum(jnp.maximum(rho, rm), hi)
        rf = (rhoc - rm) / dr
        ri = rf.astype(i32)
        rfr = rf - ri.astype(f32)
        rn = jnp.minimum(ri + 1, n_rho - 1)
        eb = t * n_rho
        F0 = plsc.load_gather(tab_e, [eb + ri])
        F1 = plsc.load_gather(tab_e, [eb + rn])
        G0 = plsc.load_gather(tab_ed, [eb + ri])
        G1 = plsc.load_gather(tab_ed, [eb + rn])
        fpv = G0 + rfr * (G1 - G0)
        plsc.store_scatter(fpb, [lids], fpv)
        m = jnp.where(s0 + lids < n_real, 1.0, 0.0)
        return acc + (F0 + rfr * (F1 - F0)) * m

    acc = lax.fori_loop(0, apw // LAN, step, jnp.zeros((LAN,), f32))
    pltpu.sync_copy(fpb, fp_out.at[pl.ds(s0, apw)])
    ev[...] = acc
    pltpu.sync_copy(ev, eemb_part.at[pl.ds(wid * 16, 16)])


def _body_c(per_w, staged, rowr, colr, fp_in, zeros1, f_part, vir_part,
            st, row2d, col2d, fprb, fpcb, fxb, fyb, fzb, ev,
            sh_fx, sh_fy, sh_fz, sh_fp, sem1, sem2):
    cid = lax.axis_index("c")
    sid = lax.axis_index("s")
    wid = cid * NS + sid
    npad = f_part.shape[0] // (NC * 3)
    sl1 = npad // NS

    pltpu.sync_copy(zeros1, sh_fx.at[pl.ds(sid * sl1, sl1)])
    pltpu.sync_copy(zeros1, sh_fy.at[pl.ds(sid * sl1, sl1)])
    pltpu.sync_copy(zeros1, sh_fz.at[pl.ds(sid * sl1, sl1)])
    pltpu.sync_copy(fp_in.at[pl.ds(sid * sl1, sl1)],
                    sh_fp.at[pl.ds(sid * sl1, sl1)])
    plsc.subcore_barrier()

    iot = lax.iota(i32, LAN)
    n_chunks = per_w // CHK
    gpw = per_w // 128

    def chunk(t, acc):
        g0 = wid * gpw + t * GRP
        e0 = wid * per_w + t * CHK
        c1 = pltpu.async_copy(rowr.at[pl.ds(g0, GRP), :], row2d, sem1)
        c2 = pltpu.async_copy(colr.at[pl.ds(g0, GRP), :], col2d, sem1)
        c1.wait()
        c2.wait()
        descs = [pltpu.async_copy(staged.at[:, pl.ds(e0, CHK)], st, sem1)]
        for g in range(GRP):
            descs.append(pltpu.async_copy(
                sh_fp.at[row2d.at[g]], fprb.at[pl.ds(g * 128, 128)], sem2))
            descs.append(pltpu.async_copy(
                sh_fp.at[col2d.at[g]], fpcb.at[pl.ds(g * 128, 128)], sem2))
        for d in descs:
            d.wait()

        def step(j4, acc):
          for q in range(4):
            j = j4 * 4 + q
            lids = j * LAN + iot
            fpr = plsc.load_gather(fprb, [lids])
            fpc = plsc.load_gather(fpcb, [lids])
            a = plsc.load_gather(st, [_full(0), lids])
            b = plsc.load_gather(st, [_full(1), lids])
            cc = plsc.load_gather(st, [_full(2), lids])
            gx = plsc.load_gather(st, [_full(3), lids])
            gy = plsc.load_gather(st, [_full(4), lids])
            gz = plsc.load_gather(st, [_full(5), lids])
            r = plsc.load_gather(st, [_full(6), lids])
            fpair = fpr * a + fpc * b + cc
            acc = acc + fpair * r
            plsc.store_scatter(fxb, [lids], -(fpair * gx))
            plsc.store_scatter(fyb, [lids], -(fpair * gy))
            plsc.store_scatter(fzb, [lids], -(fpair * gz))
          return acc

        acc = lax.fori_loop(0, CHK // LAN // 4, step, acc)
        adds = []
        for g in range(GRP):
            adds.append(pltpu.async_copy(fxb.at[pl.ds(g * 128, 128)],
                                         sh_fx.at[row2d.at[g]], sem2, add=True))
            adds.append(pltpu.async_copy(fyb.at[pl.ds(g * 128, 128)],
                                         sh_fy.at[row2d.at[g]], sem2, add=True))
            adds.append(pltpu.async_copy(fzb.at[pl.ds(g * 128, 128)],
                                         sh_fz.at[row2d.at[g]], sem2, add=True))
        for d in adds:
            d.wait()
        return acc

    acc = lax.fori_loop(0, n_chunks, chunk, jnp.zeros((LAN,), f32))
    plsc.subcore_barrier()
    base = cid * 3 * npad + sid * sl1
    pltpu.sync_copy(sh_fx.at[pl.ds(sid * sl1, sl1)],
                    f_part.at[pl.ds(base, sl1)])
    pltpu.sync_copy(sh_fy.at[pl.ds(sid * sl1, sl1)],
                    f_part.at[pl.ds(base + npad, sl1)])
    pltpu.sync_copy(sh_fz.at[pl.ds(sid * sl1, sl1)],
                    f_part.at[pl.ds(base + 2 * npad, sl1)])
    ev[...] = acc
    pltpu.sync_copy(ev, vir_part.at[pl.ds(wid * 16, 16)])


def kernel(coords, edge_index, atom_types, density_table, density_deriv_table,
           pair_table, pair_deriv_table, embed_table, embed_deriv_table,
           embed_rho_min, embed_drho, cutoff, box_length):
    N = coords.shape[0]
    E = edge_index.shape[1]
    NT = density_table.shape[0]
    n_r = density_table.shape[1]
    n_rho = embed_table.shape[1]

    per_w = -(-E // (NW * CHK)) * CHK
    e_pad = NW * per_w
    npad = -(-N // (NW * 8)) * (NW * 8)
    sl1 = npad // NS

    cut = jnp.asarray(cutoff, f32)
    lbox = jnp.asarray(box_length, f32)
    inv_dr = 1.0 / (cut / (n_r - 1))
    rclip = cut * (1.0 - 1e-07)
    params = jnp.tile(jnp.stack([cut, lbox, inv_dr, rclip])[:, None],
                      (1, 16)).astype(f32)

    atoms16 = jnp.concatenate([coords, atom_types[:, None].astype(f32)], axis=1)
    atoms16 = jnp.pad(atoms16, ((0, npad - N), (0, 12)))
    rowr = jnp.pad(edge_index[0], (0, e_pad - E)).reshape(e_pad // 128, 128)
    colr = jnp.pad(edge_index[1], (0, e_pad - E)).reshape(e_pad // 128, 128)
    dens = density_table.reshape(-1)
    densd = density_deriv_table.reshape(-1)
    pairf = pair_table.reshape(-1)
    pairdf = pair_deriv_table.reshape(-1)
    emb = embed_table.reshape(-1)
    embd = embed_deriv_table.reshape(-1)
    rm16 = jnp.zeros((16,), f32).at[:NT].set(embed_rho_min)
    dr16 = jnp.ones((16,), f32).at[:NT].set(embed_drho)
    types_pad = jnp.pad(atom_types, (0, npad - N))
    zeros1 = jnp.zeros((sl1,), f32)

    mesh = plsc.VectorSubcoreMesh(core_axis_name="c", subcore_axis_name="s")

    ka1 = pl.kernel(
        functools.partial(_body_a1, per_w),
        out_type=jax.ShapeDtypeStruct((5, e_pad), f32),
        mesh=mesh,
        compiler_params=_SC_PARAMS,
        scratch_types=[
            pltpu.VMEM((4, 16), f32),
            pltpu.VMEM((GRP, 128), i32),
            pltpu.VMEM((GRP, 128), i32),
            pltpu.VMEM((CHK, 16), f32),
            pltpu.VMEM((CHK, 16), f32),
            pltpu.VMEM((5, CHK), f32),
            pltpu.SemaphoreType.DMA,
            pltpu.SemaphoreType.DMA,
        ],
    )
    geo = ka1(atoms16, rowr, colr, params)

    blk = 32768
    while e_pad % blk:
        blk //= 2
    r_all = pl.pallas_call(
        _tc_r_body,
        grid=(e_pad // blk,),
        in_specs=[pl.BlockSpec((5, blk), lambda i: (0, i))],
        out_specs=pl.BlockSpec((1, blk), lambda i: (0, i)),
        out_shape=jax.ShapeDtypeStruct((1, e_pad), f32),
    )(geo)

    ka2 = pl.kernel(
        functools.partial(_body_a2, n_r, NT, per_w),
        out_type=(
            jax.ShapeDtypeStruct((7, e_pad), f32),
            jax.ShapeDtypeStruct((NC * npad,), f32),
            jax.ShapeDtypeStruct((NW * 16,), f32),
        ),
        mesh=mesh,
        compiler_params=_SC_PARAMS,
        scratch_types=[
            pltpu.VMEM((NT * n_r,), f32),
            pltpu.VMEM((NT * n_r,), f32),
            pltpu.VMEM((NT * NT * n_r,), f32),
            pltpu.VMEM((NT * NT * n_r,), f32),
            pltpu.VMEM((4, 16), f32),
            pltpu.VMEM((GRP, 128), i32),
            pltpu.VMEM((5, CHK), f32),
            pltpu.VMEM((CHK,), f32),
            pltpu.VMEM((7, CHK), f32),
            pltpu.VMEM((CHK,), f32),
            pltpu.VMEM((16,), f32),
            pltpu.VMEM_SHARED((npad,), f32),
            pltpu.SemaphoreType.DMA,
            pltpu.SemaphoreType.DMA,
        ],
    )
    staged, rho_part, epair_part = ka2(geo, r_all, rowr, dens, densd,
                                       pairf, pairdf, params, zeros1)

    kb = pl.kernel(
        functools.partial(_body_b, n_rho, N),
        out_type=(
            jax.ShapeDtypeStruct((npad,), f32),
            jax.ShapeDtypeStruct((NW * 16,), f32),
        ),
        mesh=mesh,
        compiler_params=_SC_PARAMS,
        scratch_types=[
            pltpu.VMEM((NT * n_rho,), f32),
            pltpu.VMEM((NT * n_rho,), f32),
            pltpu.VMEM((npad // NW,), f32),
            pltpu.VMEM((npad // NW,), f32),
            pltpu.VMEM((npad // NW,), i32),
            pltpu.VMEM((npad // NW,), f32),
            pltpu.VMEM((16,), f32),
            pltpu.VMEM((16,), f32),
            pltpu.VMEM((16,), f32),
            pltpu.SemaphoreType.DMA,
        ],
    )
    fp_val, eemb_part = kb(rho_part, types_pad, emb, embd, rm16, dr16)

    kc = pl.kernel(
        functools.partial(_body_c, per_w),
        out_type=(
            jax.ShapeDtypeStruct((NC * 3 * npad,), f32),
            jax.ShapeDtypeStruct((NW * 16,), f32),
        ),
        mesh=mesh,
        compiler_params=_SC_PARAMS,
        scratch_types=[
            pltpu.VMEM((7, CHK), f32),
            pltpu.VMEM((GRP, 128), i32),
            pltpu.VMEM((GRP, 128), i32),
            pltpu.VMEM((CHK,), f32),
            pltpu.VMEM((CHK,), f32),
            pltpu.VMEM((CHK,), f32),
            pltpu.VMEM((CHK,), f32),
            pltpu.VMEM((CHK,), f32),
            pltpu.VMEM((16,), f32),
            pltpu.VMEM_SHARED((npad,), f32),
            pltpu.VMEM_SHARED((npad,), f32),
            pltpu.VMEM_SHARED((npad,), f32),
            pltpu.VMEM_SHARED((npad,), f32),
            pltpu.SemaphoreType.DMA,
            pltpu.SemaphoreType.DMA,
        ],
    )
    f_part, vir_part = kc(staged, rowr, colr, fp_val, zeros1)

    energy = jnp.sum(eemb_part) + 0.5 * jnp.sum(epair_part)
    virial = -0.5 * jnp.sum(vir_part)
    fp3 = f_part.reshape(NC, 3, npad)
    forces = (fp3[0] + fp3[1]).T[:N]
    return energy, forces, virial


# speedup vs baseline: 145.6804x; 1.0917x over previous
import functools

import jax
import jax.numpy as jnp
from jax import lax
from jax.experimental import pallas as pl
from jax.experimental.pallas import tpu as pltpu
from jax.experimental.pallas import tpu_sc as plsc

f32 = jnp.float32
i32 = jnp.int32

NC = 2
NS = 16
NW = NC * NS
LAN = 16
CHK = 1024
GRP = CHK // 128

_SC_PARAMS = pltpu.CompilerParams(needs_layout_passes=False,
                                  use_tc_tiling_on_sc=False)


def _full(v, dtype=i32):
    return jnp.full((LAN,), v, dtype=dtype)


def _body_a1(per_w, atoms16, rowr, colr, params, geo,
             pv, row2d, col2d, arow, acol, sbuf, sh_at, sem1, sem2):
    cid = lax.axis_index("c")
    sid = lax.axis_index("s")
    wid = cid * NS + sid
    npad = atoms16.shape[0]
    sl1 = npad // NS
    pltpu.sync_copy(params, pv)
    pltpu.sync_copy(atoms16.at[pl.ds(sid * sl1, sl1), :],
                    sh_at.at[pl.ds(sid * sl1, sl1), :])
    plsc.subcore_barrier()
    iot = lax.iota(i32, LAN)
    lbox = pv[1]
    gpw = per_w // 128

    def chunk(t, carry):
        g0 = wid * gpw + t * GRP
        e0 = wid * per_w + t * CHK
        c1 = pltpu.async_copy(rowr.at[pl.ds(g0, GRP), :], row2d, sem1)
        c2 = pltpu.async_copy(colr.at[pl.ds(g0, GRP), :], col2d, sem1)
        c1.wait()
        c2.wait()
        descs = []
        for g in range(GRP):
            descs.append(pltpu.async_copy(
                sh_at.at[row2d.at[g]], arow.at[pl.ds(g * 128, 128), :], sem2))
            descs.append(pltpu.async_copy(
                sh_at.at[col2d.at[g]], acol.at[pl.ds(g * 128, 128), :], sem2))
        for d in descs:
            d.wait()

        def step(j4, carry):
          for q in range(4):
            j = j4 * 4 + q
            lids = j * LAN + iot
            xr = plsc.load_gather(arow, [lids, _full(0)])
            yr = plsc.load_gather(arow, [lids, _full(1)])
            zr = plsc.load_gather(arow, [lids, _full(2)])
            tr = plsc.load_gather(arow, [lids, _full(3)])
            xc = plsc.load_gather(acol, [lids, _full(0)])
            yc = plsc.load_gather(acol, [lids, _full(1)])
            zc = plsc.load_gather(acol, [lids, _full(2)])
            tc = plsc.load_gather(acol, [lids, _full(3)])

            def mimage(d):
                u = d / lbox
                s = jnp.where(u > 0.5, 1.0, jnp.where(u < -0.5, -1.0, 0.0))
                return d - s * lbox

            plsc.store_scatter(sbuf, [_full(0), lids], mimage(xr - xc))
            plsc.store_scatter(sbuf, [_full(1), lids], mimage(yr - yc))
            plsc.store_scatter(sbuf, [_full(2), lids], mimage(zr - zc))
            plsc.store_scatter(sbuf, [_full(3), lids], tr)
            plsc.store_scatter(sbuf, [_full(4), lids], tc)
          return carry

        carry = lax.fori_loop(0, CHK // LAN // 4, step, carry)
        pltpu.sync_copy(sbuf, geo.at[:, pl.ds(e0, CHK)])
        return carry

    lax.fori_loop(0, per_w // CHK, chunk, jnp.zeros((LAN,), f32))


def _tc_r_body(geo_ref, r_ref):
    dx = geo_ref[0, :]
    dy = geo_ref[1, :]
    dz = geo_ref[2, :]
    r_ref[0, :] = jnp.sqrt(dx * dx + dy * dy + dz * dz + 1e-12)


def _body_a2(n_r, nt, per_w, geo, r_all, rowr, dens, densd, pairf, pairdf,
             params, zeros1, staged, rho_part, epair_part,
             tab_d, tab_dd, tab_p, tab_pd, pv, row2d, gbuf, rbuf,
             sbuf, fcolb, ev, sh_rho, sem1, sem2):
    cid = lax.axis_index("c")
    sid = lax.axis_index("s")
    wid = cid * NS + sid
    npad = rho_part.shape[0] // NC
    sl1 = npad // NS

    pltpu.sync_copy(dens, tab_d)
    pltpu.sync_copy(densd, tab_dd)
    pltpu.sync_copy(pairf, tab_p)
    pltpu.sync_copy(pairdf, tab_pd)
    pltpu.sync_copy(params, pv)
    pltpu.sync_copy(zeros1, sh_rho.at[pl.ds(sid * sl1, sl1)])
    plsc.subcore_barrier()

    iot = lax.iota(i32, LAN)
    cutv = pv[0]
    invdr = pv[2]
    rclip = pv[3]
    n_chunks = per_w // CHK
    gpw = per_w // 128

    def chunk(t, acc):
        g0 = wid * gpw + t * GRP
        e0 = wid * per_w + t * CHK
        c1 = pltpu.async_copy(rowr.at[pl.ds(g0, GRP), :], row2d, sem1)
        c2 = pltpu.async_copy(geo.at[:, pl.ds(e0, CHK)], gbuf, sem1)
        c3 = pltpu.async_copy(r_all.at[0, pl.ds(e0, CHK)], rbuf, sem1)
        c1.wait()
        c2.wait()
        c3.wait()

        def step(j4, acc):
          for q in range(4):
            j = j4 * 4 + q
            lids = j * LAN + iot
            dx = plsc.load_gather(gbuf, [_full(0), lids])
            dy = plsc.load_gather(gbuf, [_full(1), lids])
            dz = plsc.load_gather(gbuf, [_full(2), lids])
            tr = plsc.load_gather(gbuf, [_full(3), lids])
            tc = plsc.load_gather(gbuf, [_full(4), lids])
            r = plsc.load_gather(rbuf, [lids])
            wl = (r < cutv) & (r > 1e-6)
            wf = jnp.where(wl, 1.0, 0.0)
            rc = jnp.minimum(r, rclip)
            fidx = rc * invdr
            idxi = fidx.astype(i32)
            frac = fidx - idxi.astype(f32)
            nidxi = jnp.minimum(idxi + 1, n_r - 1)
            tci = tc.astype(i32)
            tri = tr.astype(i32)
            dbc = tci * n_r
            dbr = tri * n_r
            pb = (tri * nt + tci) * n_r

            v0 = plsc.load_gather(tab_d, [dbc + idxi])
            v1 = plsc.load_gather(tab_d, [dbc + nidxi])
            fcol = (v0 + frac * (v1 - v0)) * wf
            p0 = plsc.load_gather(tab_p, [pb + idxi])
            p1 = plsc.load_gather(tab_p, [pb + nidxi])
            acc = acc + (p0 + frac * (p1 - p0)) * wf
            q0 = plsc.load_gather(tab_pd, [pb + idxi])
            q1 = plsc.load_gather(tab_pd, [pb + nidxi])
            cval = (q0 + frac * (q1 - q0)) * wf
            a0 = plsc.load_gather(tab_dd, [dbc + idxi])
            a1 = plsc.load_gather(tab_dd, [dbc + nidxi])
            aval = (a0 + frac * (a1 - a0)) * wf
            b0 = plsc.load_gather(tab_dd, [dbr + idxi])
            b1 = plsc.load_gather(tab_dd, [dbr + nidxi])
            bval = (b0 + frac * (b1 - b0)) * wf
            rs = jnp.where(r > 1e-6, r, 1.0)
            plsc.store_scatter(sbuf, [_full(0), lids], aval)
            plsc.store_scatter(sbuf, [_full(1), lids], bval)
            plsc.store_scatter(sbuf, [_full(2), lids], cval)
            plsc.store_scatter(sbuf, [_full(3), lids], dx / rs)
            plsc.store_scatter(sbuf, [_full(4), lids], dy / rs)
            plsc.store_scatter(sbuf, [_full(5), lids], dz / rs)
            plsc.store_scatter(sbuf, [_full(6), lids], r)
            plsc.store_scatter(fcolb, [lids], fcol)
          return acc

        acc = lax.fori_loop(0, CHK // LAN // 4, step, acc)
        wcp = pltpu.async_copy(sbuf, staged.at[:, pl.ds(e0, CHK)], sem1)
        adds = [pltpu.async_copy(fcolb.at[pl.ds(g * 128, 128)],
                                 sh_rho.at[row2d.at[g]], sem2, add=True)
                for g in range(GRP)]
        for d in adds:
            d.wait()
        wcp.wait()
        return acc

    acc = lax.fori_loop(0, n_chunks, chunk, jnp.zeros((LAN,), f32))
    plsc.subcore_barrier()
    pltpu.sync_copy(sh_rho.at[pl.ds(sid * sl1, sl1)],
                    rho_part.at[pl.ds(cid * npad + sid * sl1, sl1)])
    ev[...] = acc
    pltpu.sync_copy(ev, epair_part.at[pl.ds(wid * 16, 16)])


def _body_b(n_rho, n_real, rho_part, types_pad, emb, embd, rm16, dr16,
            fp_out, eemb_part, tab_e, tab_ed, r0b, r1b, tyb, fpb,
            rmv, drv, ev, sem):
    cid = lax.axis_index("c")
    sid = lax.axis_index("s")
    wid = cid * NS + sid
    npad = fp_out.shape[0]
    apw = npad // NW
    s0 = wid * apw

    pltpu.sync_copy(emb, tab_e)
    pltpu.sync_copy(embd, tab_ed)
    pltpu.sync_copy(rm16, rmv)
    pltpu.sync_copy(dr16, drv)
    pltpu.sync_copy(rho_part.at[pl.ds(s0, apw)], r0b)
    pltpu.sync_copy(rho_part.at[pl.ds(npad + s0, apw)], r1b)
    pltpu.sync_copy(types_pad.at[pl.ds(s0, apw)], tyb)

    iot = lax.iota(i32, LAN)
    khi = float((n_rho - 1) * (1.0 - 1e-07))

    def step(j, acc):
        lids = j * LAN + iot
        rho = plsc.load_gather(r0b, [lids]) + plsc.load_gather(r1b, [lids])
        t = plsc.load_gather(tyb, [lids])
        rm = plsc.load_gather(rmv, [t])
        dr = plsc.load_gather(drv, [t])
        hi = rm + dr * khi
        rhoc = jnp.minimum(jnp.maximum(rho, rm), hi)
        rf = (rhoc - rm) / dr
        ri = rf.astype(i32)
        rfr = rf - ri.astype(f32)
        rn = jnp.minimum(ri + 1, n_rho - 1)
        eb = t * n_rho
        F0 = plsc.load_gather(tab_e, [eb + ri])
        F1 = plsc.load_gather(tab_e, [eb + rn])
        G0 = plsc.load_gather(tab_ed, [eb + ri])
        G1 = plsc.load_gather(tab_ed, [eb + rn])
        fpv = G0 + rfr * (G1 - G0)
        plsc.store_scatter(fpb, [lids], fpv)
        m = jnp.where(s0 + lids < n_real, 1.0, 0.0)
        return acc + (F0 + rfr * (F1 - F0)) * m

    acc = lax.fori_loop(0, apw // LAN, step, jnp.zeros((LAN,), f32))
    pltpu.sync_copy(fpb, fp_out.at[pl.ds(s0, apw)])
    ev[...] = acc
    pltpu.sync_copy(ev, eemb_part.at[pl.ds(wid * 16, 16)])


def _body_c(per_w, staged, rowr, colr, fp_in, zeros1, f_part, vir_part,
            st, row2d, col2d, fprb, fpcb, fxb, fyb, fzb, ev,
            sh_fx, sh_fy, sh_fz, sh_fp, sem1, sem2):
    cid = lax.axis_index("c")
    sid = lax.axis_index("s")
    wid = cid * NS + sid
    npad = f_part.shape[0] // (NC * 3)
    sl1 = npad // NS

    pltpu.sync_copy(zeros1, sh_fx.at[pl.ds(sid * sl1, sl1)])
    pltpu.sync_copy(zeros1, sh_fy.at[pl.ds(sid * sl1, sl1)])
    pltpu.sync_copy(zeros1, sh_fz.at[pl.ds(sid * sl1, sl1)])
    pltpu.sync_copy(fp_in.at[pl.ds(sid * sl1, sl1)],
                    sh_fp.at[pl.ds(sid * sl1, sl1)])
    plsc.subcore_barrier()

    iot = lax.iota(i32, LAN)
    n_chunks = per_w // CHK
    gpw = per_w // 128

    def chunk(t, acc):
        g0 = wid * gpw + t * GRP
        e0 = wid * per_w + t * CHK
        c1 = pltpu.async_copy(rowr.at[pl.ds(g0, GRP), :], row2d, sem1)
        c2 = pltpu.async_copy(colr.at[pl.ds(g0, GRP), :], col2d, sem1)
        c1.wait()
        c2.wait()
        descs = [pltpu.async_copy(staged.at[:, pl.ds(e0, CHK)], st, sem1)]
        for g in range(GRP):
            descs.append(pltpu.async_copy(
                sh_fp.at[row2d.at[g]], fprb.at[pl.ds(g * 128, 128)], sem2))
            descs.append(pltpu.async_copy(
                sh_fp.at[col2d.at[g]], fpcb.at[pl.ds(g * 128, 128)], sem2))
        for d in descs:
            d.wait()

        def step(j4, acc):
          for q in range(4):
            j = j4 * 4 + q
            lids = j * LAN + iot
            fpr = plsc.load_gather(fprb, [lids])
            fpc = plsc.load_gather(fpcb, [lids])
            a = plsc.load_gather(st, [_full(0), lids])
            b = plsc.load_gather(st, [_full(1), lids])
            cc = plsc.load_gather(st, [_full(2), lids])
            gx = plsc.load_gather(st, [_full(3), lids])
            gy = plsc.load_gather(st, [_full(4), lids])
            gz = plsc.load_gather(st, [_full(5), lids])
            r = plsc.load_gather(st, [_full(6), lids])
            fpair = fpr * a + fpc * b + cc
            acc = acc + fpair * r
            plsc.store_scatter(fxb, [lids], -(fpair * gx))
            plsc.store_scatter(fyb, [lids], -(fpair * gy))
            plsc.store_scatter(fzb, [lids], -(fpair * gz))
          return acc

        acc = lax.fori_loop(0, CHK // LAN // 4, step, acc)
        adds = []
        for g in range(GRP):
            adds.append(pltpu.async_copy(fxb.at[pl.ds(g * 128, 128)],
                                         sh_fx.at[row2d.at[g]], sem2, add=True))
            adds.append(pltpu.async_copy(fyb.at[pl.ds(g * 128, 128)],
                                         sh_fy.at[row2d.at[g]], sem2, add=True))
            adds.append(pltpu.async_copy(fzb.at[pl.ds(g * 128, 128)],
                                         sh_fz.at[row2d.at[g]], sem2, add=True))
        for d in adds:
            d.wait()
        return acc

    acc = lax.fori_loop(0, n_chunks, chunk, jnp.zeros((LAN,), f32))
    plsc.subcore_barrier()
    base = cid * 3 * npad + sid * sl1
    pltpu.sync_copy(sh_fx.at[pl.ds(sid * sl1, sl1)],
                    f_part.at[pl.ds(base, sl1)])
    pltpu.sync_copy(sh_fy.at[pl.ds(sid * sl1, sl1)],
                    f_part.at[pl.ds(base + npad, sl1)])
    pltpu.sync_copy(sh_fz.at[pl.ds(sid * sl1, sl1)],
                    f_part.at[pl.ds(base + 2 * npad, sl1)])
    ev[...] = acc
    pltpu.sync_copy(ev, vir_part.at[pl.ds(wid * 16, 16)])


def kernel(coords, edge_index, atom_types, density_table, density_deriv_table,
           pair_table, pair_deriv_table, embed_table, embed_deriv_table,
           embed_rho_min, embed_drho, cutoff, box_length):
    N = coords.shape[0]
    E = edge_index.shape[1]
    NT = density_table.shape[0]
    n_r = density_table.shape[1]
    n_rho = embed_table.shape[1]

    per_w = -(-E // (NW * CHK)) * CHK
    e_pad = NW * per_w
    npad = -(-N // (NW * 8)) * (NW * 8)
    sl1 = npad // NS

    cut = jnp.asarray(cutoff, f32)
    lbox = jnp.asarray(box_length, f32)
    inv_dr = 1.0 / (cut / (n_r - 1))
    rclip = cut * (1.0 - 1e-07)
    params = jnp.tile(jnp.stack([cut, lbox, inv_dr, rclip])[:, None],
                      (1, 16)).astype(f32)

    atoms16 = jnp.concatenate([coords, atom_types[:, None].astype(f32)], axis=1)
    atoms16 = jnp.pad(atoms16, ((0, npad - N), (0, 12)))
    rowr = jnp.pad(edge_index[0], (0, e_pad - E)).reshape(e_pad // 128, 128)
    colr = jnp.pad(edge_index[1], (0, e_pad - E)).reshape(e_pad // 128, 128)
    dens = density_table.reshape(-1)
    densd = density_deriv_table.reshape(-1)
    pairf = pair_table.reshape(-1)
    pairdf = pair_deriv_table.reshape(-1)
    emb = embed_table.reshape(-1)
    embd = embed_deriv_table.reshape(-1)
    rm16 = jnp.zeros((16,), f32).at[:NT].set(embed_rho_min)
    dr16 = jnp.ones((16,), f32).at[:NT].set(embed_drho)
    types_pad = jnp.pad(atom_types, (0, npad - N))
    zeros1 = jnp.zeros((sl1,), f32)

    mesh = plsc.VectorSubcoreMesh(core_axis_name="c", subcore_axis_name="s")

    ka1 = pl.kernel(
        functools.partial(_body_a1, per_w),
        out_type=jax.ShapeDtypeStruct((5, e_pad), f32),
        mesh=mesh,
        compiler_params=_SC_PARAMS,
        scratch_types=[
            pltpu.VMEM((4, 16), f32),
            pltpu.VMEM((GRP, 128), i32),
            pltpu.VMEM((GRP, 128), i32),
            pltpu.VMEM((CHK, 16), f32),
            pltpu.VMEM((CHK, 16), f32),
            pltpu.VMEM((5, CHK), f32),
            pltpu.VMEM_SHARED((npad, 16), f32),
            pltpu.SemaphoreType.DMA,
            pltpu.SemaphoreType.DMA,
        ],
    )
    geo = ka1(atoms16, rowr, colr, params)

    blk = 32768
    while e_pad % blk:
        blk //= 2
    r_all = pl.pallas_call(
        _tc_r_body,
        grid=(e_pad // blk,),
        in_specs=[pl.BlockSpec((5, blk), lambda i: (0, i))],
        out_specs=pl.BlockSpec((1, blk), lambda i: (0, i)),
        out_shape=jax.ShapeDtypeStruct((1, e_pad), f32),
    )(geo)

    ka2 = pl.kernel(
        functools.partial(_body_a2, n_r, NT, per_w),
        out_type=(
            jax.ShapeDtypeStruct((7, e_pad), f32),
            jax.ShapeDtypeStruct((NC * npad,), f32),
            jax.ShapeDtypeStruct((NW * 16,), f32),
        ),
        mesh=mesh,
        compiler_params=_SC_PARAMS,
        scratch_types=[
            pltpu.VMEM((NT * n_r,), f32),
            pltpu.VMEM((NT * n_r,), f32),
            pltpu.VMEM((NT * NT * n_r,), f32),
            pltpu.VMEM((NT * NT * n_r,), f32),
            pltpu.VMEM((4, 16), f32),
            pltpu.VMEM((GRP, 128), i32),
            pltpu.VMEM((5, CHK), f32),
            pltpu.VMEM((CHK,), f32),
            pltpu.VMEM((7, CHK), f32),
            pltpu.VMEM((CHK,), f32),
            pltpu.VMEM((16,), f32),
            pltpu.VMEM_SHARED((npad,), f32),
            pltpu.SemaphoreType.DMA,
            pltpu.SemaphoreType.DMA,
        ],
    )
    staged, rho_part, epair_part = ka2(geo, r_all, rowr, dens, densd,
                                       pairf, pairdf, params, zeros1)

    kb = pl.kernel(
        functools.partial(_body_b, n_rho, N),
        out_type=(
            jax.ShapeDtypeStruct((npad,), f32),
            jax.ShapeDtypeStruct((NW * 16,), f32),
        ),
        mesh=mesh,
        compiler_params=_SC_PARAMS,
        scratch_types=[
            pltpu.VMEM((NT * n_rho,), f32),
            pltpu.VMEM((NT * n_rho,), f32),
            pltpu.VMEM((npad // NW,), f32),
            pltpu.VMEM((npad // NW,), f32),
            pltpu.VMEM((npad // NW,), i32),
            pltpu.VMEM((npad // NW,), f32),
            pltpu.VMEM((16,), f32),
            pltpu.VMEM((16,), f32),
            pltpu.VMEM((16,), f32),
            pltpu.SemaphoreType.DMA,
        ],
    )
    fp_val, eemb_part = kb(rho_part, types_pad, emb, embd, rm16, dr16)

    kc = pl.kernel(
        functools.partial(_body_c, per_w),
        out_type=(
            jax.ShapeDtypeStruct((NC * 3 * npad,), f32),
            jax.ShapeDtypeStruct((NW * 16,), f32),
        ),
        mesh=mesh,
        compiler_params=_SC_PARAMS,
        scratch_types=[
            pltpu.VMEM((7, CHK), f32),
            pltpu.VMEM((GRP, 128), i32),
            pltpu.VMEM((GRP, 128), i32),
            pltpu.VMEM((CHK,), f32),
            pltpu.VMEM((CHK,), f32),
            pltpu.VMEM((CHK,), f32),
            pltpu.VMEM((CHK,), f32),
            pltpu.VMEM((CHK,), f32),
            pltpu.VMEM((16,), f32),
            pltpu.VMEM_SHARED((npad,), f32),
            pltpu.VMEM_SHARED((npad,), f32),
            pltpu.VMEM_SHARED((npad,), f32),
            pltpu.VMEM_SHARED((npad,), f32),
            pltpu.SemaphoreType.DMA,
            pltpu.SemaphoreType.DMA,
        ],
    )
    f_part, vir_part = kc(staged, rowr, colr, fp_val, zeros1)

    energy = jnp.sum(eemb_part) + 0.5 * jnp.sum(epair_part)
    virial = -0.5 * jnp.sum(vir_part)
    fp3 = f_part.reshape(NC, 3, npad)
    forces = (fp3[0] + fp3[1]).T[:N]
    return energy, forces, virial
